# Initial kernel scaffold; baseline (speedup 1.0000x reference)
import jax
import jax.numpy as jnp
from jax.experimental import pallas as pl


def kernel(x, edge_index, edge_label_index, W_l1, W_r1, b1, W_l2, W_r2, b2, Ws, bs):
    raise NotImplementedError("write your pallas kernel here")


# speedup vs baseline: 6.2191x; 6.2191x over previous
import functools

import jax
import jax.numpy as jnp
from jax import lax
from jax.experimental import pallas as pl
from jax.experimental.pallas import tpu as pltpu
from jax.experimental.pallas import tpu_sc as plsc

N = 10000
E = 320000
LBL = 100000
NC, NS = 2, 16
NW = NC * NS
EB = 128
J = 80
E_PAD = NW * J * EB
LT = 3136
L_PAD = NW * LT
NPAD = 10112
RZ = NPAD // NS
CW = 8

@functools.cache
def _mesh():
    return plsc.VectorSubcoreMesh(core_axis_name="c", subcore_axis_name="s",
                                  num_cores=NC, num_subcores=NS)


def _sc_agg(u, src_rs, dst_rs, z64, z8, ones8, with_cnt):
    out_type = [jax.ShapeDtypeStruct((NPAD, 64), jnp.float32),
                jax.ShapeDtypeStruct((NPAD, 64), jnp.float32)]
    scratch = [pltpu.VMEM((J, EB), jnp.int32),
               pltpu.VMEM((J, EB), jnp.int32),
               pltpu.VMEM((EB, 64), jnp.float32),
               pltpu.VMEM_SHARED((NPAD, 64), jnp.float32)]
    if with_cnt:
        out_type += [jax.ShapeDtypeStruct((NPAD, CW), jnp.float32),
                     jax.ShapeDtypeStruct((NPAD, CW), jnp.float32)]
        scratch += [pltpu.VMEM((EB, CW), jnp.float32),
                    pltpu.VMEM_SHARED((NPAD, CW), jnp.float32)]

    def body(*refs):
        if with_cnt:
            (u_hbm, src_hbm, dst_hbm, z64_hbm, z8_hbm, ones_hbm,
             agg0_hbm, agg1_hbm, cnt0_hbm, cnt1_hbm,
             sidx, didx, rows, acc_sh, ones_v, cnt_sh) = refs
        else:
            (u_hbm, src_hbm, dst_hbm, z64_hbm,
             agg0_hbm, agg1_hbm,
             sidx, didx, rows, acc_sh) = refs
        cid = lax.axis_index("c")
        sid = lax.axis_index("s")
        wid = cid * NS + sid

        pltpu.sync_copy(z64_hbm.at[pl.ds(sid * RZ, RZ)],
                        acc_sh.at[pl.ds(sid * RZ, RZ)])
        if with_cnt:
            pltpu.sync_copy(z8_hbm.at[pl.ds(sid * RZ, RZ)],
                            cnt_sh.at[pl.ds(sid * RZ, RZ)])
            pltpu.sync_copy(ones_hbm, ones_v)
        pltpu.sync_copy(src_hbm.at[wid], sidx)
        pltpu.sync_copy(dst_hbm.at[wid], didx)
        plsc.subcore_barrier()

        def step(j, carry):
            pltpu.sync_copy(u_hbm.at[sidx.at[j]], rows)
            pltpu.sync_copy(rows, acc_sh.at[didx.at[j]], add=True)
            if with_cnt:
                pltpu.sync_copy(ones_v, cnt_sh.at[didx.at[j]], add=True)
            return carry

        lax.fori_loop(0, J, step, 0)
        plsc.subcore_barrier()

        rs = pl.ds(sid * RZ, RZ)

        @pl.when(cid == 0)
        def _():
            pltpu.sync_copy(acc_sh.at[rs], agg0_hbm.at[rs])
            if with_cnt:
                pltpu.sync_copy(cnt_sh.at[rs], cnt0_hbm.at[rs])

        @pl.when(cid == 1)
        def _():
            pltpu.sync_copy(acc_sh.at[rs], agg1_hbm.at[rs])
            if with_cnt:
                pltpu.sync_copy(cnt_sh.at[rs], cnt1_hbm.at[rs])

    kern = pl.kernel(body, out_type=out_type, mesh=_mesh(), scratch_types=scratch,
                     compiler_params=pltpu.CompilerParams(use_tc_tiling_on_sc=False))
    if with_cnt:
        return kern(u, src_rs, dst_rs, z64, z8, ones8)
    return kern(u, src_rs, dst_rs, z64)


def _sc_score(p, q, srcl_rs, dstl_rs):
    def body(p_hbm, q_hbm, srcl_hbm, dstl_hbm, out_hbm,
             p_v, q_v, si_v, di_v, out_v):
        cid = lax.axis_index("c")
        sid = lax.axis_index("s")
        wid = cid * NS + sid
        pltpu.sync_copy(p_hbm, p_v)
        pltpu.sync_copy(q_hbm, q_v)
        pltpu.sync_copy(srcl_hbm.at[wid], si_v)
        pltpu.sync_copy(dstl_hbm.at[wid], di_v)

        def step(t, carry):
            s = pl.ds(t * 16, 16)
            pv = plsc.load_gather(p_v, [si_v[s]])
            qv = plsc.load_gather(q_v, [di_v[s]])
            out_v[s] = pv + qv
            return carry

        lax.fori_loop(0, LT // 16, step, 0)
        pltpu.sync_copy(out_v, out_hbm.at[pl.ds(wid * LT, LT)])

    kern = pl.kernel(
        body,
        out_type=jax.ShapeDtypeStruct((L_PAD,), jnp.float32),
        mesh=_mesh(),
        scratch_types=[pltpu.VMEM((N,), jnp.float32),
                       pltpu.VMEM((N,), jnp.float32),
                       pltpu.VMEM((LT,), jnp.int32),
                       pltpu.VMEM((LT,), jnp.int32),
                       pltpu.VMEM((LT,), jnp.float32)],
        compiler_params=pltpu.CompilerParams(use_tc_tiling_on_sc=False,
                                             needs_layout_passes=False))
    return kern(p, q, srcl_rs, dstl_rs)


_BR = 1000


def _dense1(x, W_l, W_r, b):
    def body(x_ref, wl_ref, wr_ref, b_ref, u_ref, r_ref):
        xb = x_ref[...]
        u_ref[...] = jnp.dot(xb, wl_ref[...], preferred_element_type=jnp.float32)
        r_ref[...] = (jnp.dot(xb, wr_ref[...], preferred_element_type=jnp.float32)
                      + b_ref[...])

    return pl.pallas_call(
        body,
        grid=(N // _BR,),
        in_specs=[pl.BlockSpec((_BR, 128), lambda i: (i, 0)),
                  pl.BlockSpec((128, 64), lambda i: (0, 0)),
                  pl.BlockSpec((128, 64), lambda i: (0, 0)),
                  pl.BlockSpec((1, 64), lambda i: (0, 0))],
        out_specs=[pl.BlockSpec((_BR, 64), lambda i: (i, 0)),
                   pl.BlockSpec((_BR, 64), lambda i: (i, 0))],
        out_shape=[jax.ShapeDtypeStruct((N, 64), jnp.float32),
                   jax.ShapeDtypeStruct((N, 64), jnp.float32)],
    )(x, W_l, W_r, b)


def _dense2(a0, a1, c0, c1, r1, W_l, W_r, b):
    def body(a0_ref, a1_ref, c0_ref, c1_ref, r1_ref, wl_ref, wr_ref, b_ref,
             u_ref, r_ref, ci_ref):
        cnt = c0_ref[...][:, 0:1] + c1_ref[...][:, 0:1]
        ci = 1.0 / jnp.maximum(cnt, 1.0)
        h = jnp.maximum((a0_ref[...] + a1_ref[...]) * ci + r1_ref[...], 0.0)
        u_ref[...] = jnp.dot(h, wl_ref[...], preferred_element_type=jnp.float32)
        r_ref[...] = (jnp.dot(h, wr_ref[...], preferred_element_type=jnp.float32)
                      + b_ref[...])
        ci_ref[...] = ci

    return pl.pallas_call(
        body,
        grid=(N // _BR,),
        in_specs=[pl.BlockSpec((_BR, 64), lambda i: (i, 0)),
                  pl.BlockSpec((_BR, 64), lambda i: (i, 0)),
                  pl.BlockSpec((_BR, CW), lambda i: (i, 0)),
                  pl.BlockSpec((_BR, CW), lambda i: (i, 0)),
                  pl.BlockSpec((_BR, 64), lambda i: (i, 0)),
                  pl.BlockSpec((64, 64), lambda i: (0, 0)),
                  pl.BlockSpec((64, 64), lambda i: (0, 0)),
                  pl.BlockSpec((1, 64), lambda i: (0, 0))],
        out_specs=[pl.BlockSpec((_BR, 64), lambda i: (i, 0)),
                   pl.BlockSpec((_BR, 64), lambda i: (i, 0)),
                   pl.BlockSpec((_BR, 1), lambda i: (i, 0))],
        out_shape=[jax.ShapeDtypeStruct((N, 64), jnp.float32),
                   jax.ShapeDtypeStruct((N, 64), jnp.float32),
                   jax.ShapeDtypeStruct((N, 1), jnp.float32)],
    )(a0, a1, c0, c1, r1, W_l, W_r, b)


def _dense3(a0, a1, ci, r2, Wsr, bsr):
    def body(a0_ref, a1_ref, ci_ref, r2_ref, ws_ref, bs_ref, pq_ref):
        h = jnp.maximum((a0_ref[...] + a1_ref[...]) * ci_ref[...] + r2_ref[...],
                        0.0)
        pq_ref[...] = (jnp.dot(h, ws_ref[...], preferred_element_type=jnp.float32)
                       + bs_ref[...])

    return pl.pallas_call(
        body,
        grid=(N // _BR,),
        in_specs=[pl.BlockSpec((_BR, 64), lambda i: (i, 0)),
                  pl.BlockSpec((_BR, 64), lambda i: (i, 0)),
                  pl.BlockSpec((_BR, 1), lambda i: (i, 0)),
                  pl.BlockSpec((_BR, 64), lambda i: (i, 0)),
                  pl.BlockSpec((64, 2), lambda i: (0, 0)),
                  pl.BlockSpec((1, 2), lambda i: (0, 0))],
        out_specs=pl.BlockSpec((_BR, 2), lambda i: (i, 0)),
        out_shape=jax.ShapeDtypeStruct((N, 2), jnp.float32),
    )(a0, a1, ci, r2, Wsr, bsr)


def kernel(x, edge_index, edge_label_index, W_l1, W_r1, b1, W_l2, W_r2, b2,
           Ws, bs):
    src = edge_index[0].astype(jnp.int32)
    dst = edge_index[1].astype(jnp.int32)
    pad = E_PAD - E
    src_rs = jnp.concatenate([src, jnp.zeros((pad,), jnp.int32)]).reshape(
        NW, J, EB)
    dst_rs = jnp.concatenate([dst, jnp.full((pad,), N, jnp.int32)]).reshape(
        NW, J, EB)
    lpad = L_PAD - LBL
    srcl = edge_label_index[0].astype(jnp.int32)
    dstl = edge_label_index[1].astype(jnp.int32)
    srcl_rs = jnp.concatenate([srcl, jnp.zeros((lpad,), jnp.int32)]).reshape(
        NW, LT)
    dstl_rs = jnp.concatenate([dstl, jnp.zeros((lpad,), jnp.int32)]).reshape(
        NW, LT)

    z64 = jnp.zeros((NPAD, 64), jnp.float32)
    z8 = jnp.zeros((NPAD, CW), jnp.float32)
    ones8 = jnp.ones((EB, CW), jnp.float32)

    u1, r1 = _dense1(x, W_l1, W_r1, b1.reshape(1, 64))
    a0, a1, c0, c1 = _sc_agg(u1, src_rs, dst_rs, z64, z8, ones8, True)
    u2, r2, ci = _dense2(a0[:N], a1[:N], c0[:N], c1[:N], r1, W_l2, W_r2,
                         b2.reshape(1, 64))
    b0, b1_ = _sc_agg(u2, src_rs, dst_rs, z64, None, None, False)
    Wsr = jnp.concatenate([Ws[:64], Ws[64:]], axis=1)
    bsr = jnp.stack([bs[0], jnp.zeros((), jnp.float32)]).reshape(1, 2)
    pq = _dense3(b0[:N], b1_[:N], ci, r2, Wsr, bsr)
    p = pq[:, 0]
    q = pq[:, 1]
    out_pad = _sc_score(p, q, srcl_rs, dstl_rs)
    return out_pad[:LBL]


# speedup vs baseline: 7.4507x; 1.1980x over previous
import functools

import jax
import jax.numpy as jnp
from jax import lax
from jax.experimental import pallas as pl
from jax.experimental.pallas import tpu as pltpu
from jax.experimental.pallas import tpu_sc as plsc

N = 10000
E = 320000
LBL = 100000
NC, NS = 2, 16
NW = NC * NS
EB = 128
J = 80
E_PAD = NW * J * EB
LT = 3136
L_PAD = NW * LT
NPAD = 10112
RZ = NPAD // NS
CW = 8
NBUF = 4

@functools.cache
def _mesh():
    return plsc.VectorSubcoreMesh(core_axis_name="c", subcore_axis_name="s",
                                  num_cores=NC, num_subcores=NS)


def _sc_agg(u, src_rs, dst_rs, z64, z8, ones8, with_cnt):
    out_type = [jax.ShapeDtypeStruct((NPAD, 64), jnp.float32),
                jax.ShapeDtypeStruct((NPAD, 64), jnp.float32)]
    scratch = [pltpu.VMEM((J, EB), jnp.int32),
               pltpu.VMEM((J, EB), jnp.int32),
               pltpu.VMEM((NBUF, EB, 64), jnp.float32),
               pltpu.VMEM_SHARED((NPAD, 64), jnp.float32),
               *[pltpu.SemaphoreType.DMA] * NBUF]
    if with_cnt:
        out_type += [jax.ShapeDtypeStruct((NPAD, CW), jnp.float32),
                     jax.ShapeDtypeStruct((NPAD, CW), jnp.float32)]
        scratch += [pltpu.VMEM((EB, CW), jnp.float32),
                    pltpu.VMEM_SHARED((NPAD, CW), jnp.float32),
                    *[pltpu.SemaphoreType.DMA] * NBUF]

    def body(*refs):
        if with_cnt:
            (u_hbm, src_hbm, dst_hbm, z64_hbm, z8_hbm, ones_hbm,
             agg0_hbm, agg1_hbm, cnt0_hbm, cnt1_hbm,
             sidx, didx, rows, acc_sh, g0, g1, g2, g3,
             ones_v, cnt_sh, c0, c1, c2, c3) = refs
            gsem = [g0, g1, g2, g3]
            csem = [c0, c1, c2, c3]
        else:
            (u_hbm, src_hbm, dst_hbm, z64_hbm,
             agg0_hbm, agg1_hbm,
             sidx, didx, rows, acc_sh, g0, g1, g2, g3) = refs
            gsem = [g0, g1, g2, g3]
        cid = lax.axis_index("c")
        sid = lax.axis_index("s")
        wid = cid * NS + sid

        pltpu.sync_copy(z64_hbm.at[pl.ds(sid * RZ, RZ)],
                        acc_sh.at[pl.ds(sid * RZ, RZ)])
        if with_cnt:
            pltpu.sync_copy(z8_hbm.at[pl.ds(sid * RZ, RZ)],
                            cnt_sh.at[pl.ds(sid * RZ, RZ)])
            pltpu.sync_copy(ones_hbm, ones_v)
        pltpu.sync_copy(src_hbm.at[wid], sidx)
        pltpu.sync_copy(dst_hbm.at[wid], didx)
        plsc.subcore_barrier()

        for b in range(NBUF):
            pltpu.async_copy(u_hbm.at[sidx.at[b]], rows.at[b], gsem[b])

        def step(jj, carry):
            for b in range(NBUF):
                j = jj * NBUF + b
                pltpu.make_async_copy(u_hbm.at[sidx.at[j]], rows.at[b],
                                      gsem[b]).wait()
                pltpu.sync_copy(rows.at[b], acc_sh.at[didx.at[j]], add=True)
                if with_cnt:
                    pltpu.sync_copy(ones_v, cnt_sh.at[didx.at[j]], add=True)

                @pl.when(j + NBUF < J)
                def _():
                    pltpu.async_copy(u_hbm.at[sidx.at[j + NBUF]], rows.at[b],
                                     gsem[b])
            return carry

        lax.fori_loop(0, J // NBUF, step, 0)
        plsc.subcore_barrier()

        rs = pl.ds(sid * RZ, RZ)

        @pl.when(cid == 0)
        def _():
            pltpu.sync_copy(acc_sh.at[rs], agg0_hbm.at[rs])
            if with_cnt:
                pltpu.sync_copy(cnt_sh.at[rs], cnt0_hbm.at[rs])

        @pl.when(cid == 1)
        def _():
            pltpu.sync_copy(acc_sh.at[rs], agg1_hbm.at[rs])
            if with_cnt:
                pltpu.sync_copy(cnt_sh.at[rs], cnt1_hbm.at[rs])

    kern = pl.kernel(body, out_type=out_type, mesh=_mesh(), scratch_types=scratch,
                     compiler_params=pltpu.CompilerParams(use_tc_tiling_on_sc=False))
    if with_cnt:
        return kern(u, src_rs, dst_rs, z64, z8, ones8)
    return kern(u, src_rs, dst_rs, z64)


def _sc_score(p, q, srcl_rs, dstl_rs):
    def body(p_hbm, q_hbm, srcl_hbm, dstl_hbm, out_hbm,
             p_v, q_v, si_v, di_v, out_v):
        cid = lax.axis_index("c")
        sid = lax.axis_index("s")
        wid = cid * NS + sid
        pltpu.sync_copy(p_hbm, p_v)
        pltpu.sync_copy(q_hbm, q_v)
        pltpu.sync_copy(srcl_hbm.at[wid], si_v)
        pltpu.sync_copy(dstl_hbm.at[wid], di_v)

        def step(t, carry):
            s = pl.ds(t * 16, 16)
            pv = plsc.load_gather(p_v, [si_v[s]])
            qv = plsc.load_gather(q_v, [di_v[s]])
            out_v[s] = pv + qv
            return carry

        lax.fori_loop(0, LT // 16, step, 0)
        pltpu.sync_copy(out_v, out_hbm.at[pl.ds(wid * LT, LT)])

    kern = pl.kernel(
        body,
        out_type=jax.ShapeDtypeStruct((L_PAD,), jnp.float32),
        mesh=_mesh(),
        scratch_types=[pltpu.VMEM((N,), jnp.float32),
                       pltpu.VMEM((N,), jnp.float32),
                       pltpu.VMEM((LT,), jnp.int32),
                       pltpu.VMEM((LT,), jnp.int32),
                       pltpu.VMEM((LT,), jnp.float32)],
        compiler_params=pltpu.CompilerParams(use_tc_tiling_on_sc=False,
                                             needs_layout_passes=False))
    return kern(p, q, srcl_rs, dstl_rs)


_BR = 1000


def _dense1(x, W_l, W_r, b):
    def body(x_ref, wl_ref, wr_ref, b_ref, u_ref, r_ref):
        xb = x_ref[...]
        u_ref[...] = jnp.dot(xb, wl_ref[...], preferred_element_type=jnp.float32)
        r_ref[...] = (jnp.dot(xb, wr_ref[...], preferred_element_type=jnp.float32)
                      + b_ref[...])

    return pl.pallas_call(
        body,
        grid=(N // _BR,),
        in_specs=[pl.BlockSpec((_BR, 128), lambda i: (i, 0)),
                  pl.BlockSpec((128, 64), lambda i: (0, 0)),
                  pl.BlockSpec((128, 64), lambda i: (0, 0)),
                  pl.BlockSpec((1, 64), lambda i: (0, 0))],
        out_specs=[pl.BlockSpec((_BR, 64), lambda i: (i, 0)),
                   pl.BlockSpec((_BR, 64), lambda i: (i, 0))],
        out_shape=[jax.ShapeDtypeStruct((N, 64), jnp.float32),
                   jax.ShapeDtypeStruct((N, 64), jnp.float32)],
    )(x, W_l, W_r, b)


def _dense2(a0, a1, c0, c1, r1, W_l, W_r, b):
    def body(a0_ref, a1_ref, c0_ref, c1_ref, r1_ref, wl_ref, wr_ref, b_ref,
             u_ref, r_ref, ci_ref):
        cnt = c0_ref[...][:, 0:1] + c1_ref[...][:, 0:1]
        ci = 1.0 / jnp.maximum(cnt, 1.0)
        h = jnp.maximum((a0_ref[...] + a1_ref[...]) * ci + r1_ref[...], 0.0)
        u_ref[...] = jnp.dot(h, wl_ref[...], preferred_element_type=jnp.float32)
        r_ref[...] = (jnp.dot(h, wr_ref[...], preferred_element_type=jnp.float32)
                      + b_ref[...])
        ci_ref[...] = ci

    return pl.pallas_call(
        body,
        grid=(N // _BR,),
        in_specs=[pl.BlockSpec((_BR, 64), lambda i: (i, 0)),
                  pl.BlockSpec((_BR, 64), lambda i: (i, 0)),
                  pl.BlockSpec((_BR, CW), lambda i: (i, 0)),
                  pl.BlockSpec((_BR, CW), lambda i: (i, 0)),
                  pl.BlockSpec((_BR, 64), lambda i: (i, 0)),
                  pl.BlockSpec((64, 64), lambda i: (0, 0)),
                  pl.BlockSpec((64, 64), lambda i: (0, 0)),
                  pl.BlockSpec((1, 64), lambda i: (0, 0))],
        out_specs=[pl.BlockSpec((_BR, 64), lambda i: (i, 0)),
                   pl.BlockSpec((_BR, 64), lambda i: (i, 0)),
                   pl.BlockSpec((_BR, 1), lambda i: (i, 0))],
        out_shape=[jax.ShapeDtypeStruct((N, 64), jnp.float32),
                   jax.ShapeDtypeStruct((N, 64), jnp.float32),
                   jax.ShapeDtypeStruct((N, 1), jnp.float32)],
    )(a0, a1, c0, c1, r1, W_l, W_r, b)


def _dense3(a0, a1, ci, r2, Wsr, bsr):
    def body(a0_ref, a1_ref, ci_ref, r2_ref, ws_ref, bs_ref, pq_ref):
        h = jnp.maximum((a0_ref[...] + a1_ref[...]) * ci_ref[...] + r2_ref[...],
                        0.0)
        pq_ref[...] = (jnp.dot(h, ws_ref[...], preferred_element_type=jnp.float32)
                       + bs_ref[...])

    return pl.pallas_call(
        body,
        grid=(N // _BR,),
        in_specs=[pl.BlockSpec((_BR, 64), lambda i: (i, 0)),
                  pl.BlockSpec((_BR, 64), lambda i: (i, 0)),
                  pl.BlockSpec((_BR, 1), lambda i: (i, 0)),
                  pl.BlockSpec((_BR, 64), lambda i: (i, 0)),
                  pl.BlockSpec((64, 2), lambda i: (0, 0)),
                  pl.BlockSpec((1, 2), lambda i: (0, 0))],
        out_specs=pl.BlockSpec((_BR, 2), lambda i: (i, 0)),
        out_shape=jax.ShapeDtypeStruct((N, 2), jnp.float32),
    )(a0, a1, ci, r2, Wsr, bsr)


def kernel(x, edge_index, edge_label_index, W_l1, W_r1, b1, W_l2, W_r2, b2,
           Ws, bs):
    src = edge_index[0].astype(jnp.int32)
    dst = edge_index[1].astype(jnp.int32)
    pad = E_PAD - E
    src_rs = jnp.concatenate([src, jnp.zeros((pad,), jnp.int32)]).reshape(
        NW, J, EB)
    dst_rs = jnp.concatenate([dst, jnp.full((pad,), N, jnp.int32)]).reshape(
        NW, J, EB)
    lpad = L_PAD - LBL
    srcl = edge_label_index[0].astype(jnp.int32)
    dstl = edge_label_index[1].astype(jnp.int32)
    srcl_rs = jnp.concatenate([srcl, jnp.zeros((lpad,), jnp.int32)]).reshape(
        NW, LT)
    dstl_rs = jnp.concatenate([dstl, jnp.zeros((lpad,), jnp.int32)]).reshape(
        NW, LT)

    z64 = jnp.zeros((NPAD, 64), jnp.float32)
    z8 = jnp.zeros((NPAD, CW), jnp.float32)
    ones8 = jnp.ones((EB, CW), jnp.float32)

    u1, r1 = _dense1(x, W_l1, W_r1, b1.reshape(1, 64))
    a0, a1, c0, c1 = _sc_agg(u1, src_rs, dst_rs, z64, z8, ones8, True)
    u2, r2, ci = _dense2(a0[:N], a1[:N], c0[:N], c1[:N], r1, W_l2, W_r2,
                         b2.reshape(1, 64))
    b0, b1_ = _sc_agg(u2, src_rs, dst_rs, z64, None, None, False)
    Wsr = jnp.concatenate([Ws[:64], Ws[64:]], axis=1)
    bsr = jnp.stack([bs[0], jnp.zeros((), jnp.float32)]).reshape(1, 2)
    pq = _dense3(b0[:N], b1_[:N], ci, r2, Wsr, bsr)
    p = pq[:, 0]
    q = pq[:, 1]
    out_pad = _sc_score(p, q, srcl_rs, dstl_rs)
    return out_pad[:LBL]


# speedup vs baseline: 7.4578x; 1.0010x over previous
import functools

import jax
import jax.numpy as jnp
from jax import lax
from jax.experimental import pallas as pl
from jax.experimental.pallas import tpu as pltpu
from jax.experimental.pallas import tpu_sc as plsc

N = 10000
E = 320000
LBL = 100000
NC, NS = 2, 16
NW = NC * NS
EB = 128
J = 80
E_PAD = NW * J * EB
LT = 3136
L_PAD = NW * LT
NPAD = 10112
RZ = NPAD // NS
CW = 8
NBUF = 4

@functools.cache
def _mesh():
    return plsc.VectorSubcoreMesh(core_axis_name="c", subcore_axis_name="s",
                                  num_cores=NC, num_subcores=NS)


def _sc_agg(u, src_rs, dst_rs, z64, z8, ones8, with_cnt):
    out_type = [jax.ShapeDtypeStruct((NPAD, 64), jnp.float32),
                jax.ShapeDtypeStruct((NPAD, 64), jnp.float32)]
    scratch = [pltpu.VMEM((J, EB), jnp.int32),
               pltpu.VMEM((J, EB), jnp.int32),
               pltpu.VMEM((NBUF, EB, 64), jnp.float32),
               pltpu.VMEM_SHARED((NPAD, 64), jnp.float32),
               *[pltpu.SemaphoreType.DMA] * NBUF]
    if with_cnt:
        out_type += [jax.ShapeDtypeStruct((NPAD, CW), jnp.float32),
                     jax.ShapeDtypeStruct((NPAD, CW), jnp.float32)]
        scratch += [pltpu.VMEM((EB, CW), jnp.float32),
                    pltpu.VMEM_SHARED((NPAD, CW), jnp.float32),
                    *[pltpu.SemaphoreType.DMA] * NBUF]

    def body(*refs):
        if with_cnt:
            (u_hbm, src_hbm, dst_hbm, z64_hbm, z8_hbm, ones_hbm,
             agg0_hbm, agg1_hbm, cnt0_hbm, cnt1_hbm,
             sidx, didx, rows, acc_sh, g0, g1, g2, g3,
             ones_v, cnt_sh, c0, c1, c2, c3) = refs
            gsem = [g0, g1, g2, g3]
            csem = [c0, c1, c2, c3]
        else:
            (u_hbm, src_hbm, dst_hbm, z64_hbm,
             agg0_hbm, agg1_hbm,
             sidx, didx, rows, acc_sh, g0, g1, g2, g3) = refs
            gsem = [g0, g1, g2, g3]
        cid = lax.axis_index("c")
        sid = lax.axis_index("s")
        wid = cid * NS + sid

        pltpu.sync_copy(z64_hbm.at[pl.ds(sid * RZ, RZ)],
                        acc_sh.at[pl.ds(sid * RZ, RZ)])
        if with_cnt:
            pltpu.sync_copy(z8_hbm.at[pl.ds(sid * RZ, RZ)],
                            cnt_sh.at[pl.ds(sid * RZ, RZ)])
            pltpu.sync_copy(ones_hbm, ones_v)
        pltpu.sync_copy(src_hbm.at[wid], sidx)
        pltpu.sync_copy(dst_hbm.at[wid], didx)
        plsc.subcore_barrier()

        for b in range(NBUF):
            pltpu.async_copy(u_hbm.at[sidx.at[b]], rows.at[b], gsem[b])

        def step(jj, carry):
            for b in range(NBUF):
                j = jj * NBUF + b
                pltpu.make_async_copy(u_hbm.at[sidx.at[j]], rows.at[b],
                                      gsem[b]).wait()
                pltpu.sync_copy(rows.at[b], acc_sh.at[didx.at[j]], add=True)
                if with_cnt:
                    @pl.when(j >= NBUF)
                    def _():
                        pltpu.make_async_copy(ones_v, cnt_sh.at[didx.at[0]],
                                              csem[b]).wait()
                    pltpu.async_copy(ones_v, cnt_sh.at[didx.at[j]], csem[b],
                                     add=True)

                @pl.when(j + NBUF < J)
                def _():
                    pltpu.async_copy(u_hbm.at[sidx.at[j + NBUF]], rows.at[b],
                                     gsem[b])
            return carry

        lax.fori_loop(0, J // NBUF, step, 0)
        if with_cnt:
            for b in range(NBUF):
                pltpu.make_async_copy(ones_v, cnt_sh.at[didx.at[0]],
                                      csem[b]).wait()
        plsc.subcore_barrier()

        rs = pl.ds(sid * RZ, RZ)

        @pl.when(cid == 0)
        def _():
            pltpu.sync_copy(acc_sh.at[rs], agg0_hbm.at[rs])
            if with_cnt:
                pltpu.sync_copy(cnt_sh.at[rs], cnt0_hbm.at[rs])

        @pl.when(cid == 1)
        def _():
            pltpu.sync_copy(acc_sh.at[rs], agg1_hbm.at[rs])
            if with_cnt:
                pltpu.sync_copy(cnt_sh.at[rs], cnt1_hbm.at[rs])

    kern = pl.kernel(body, out_type=out_type, mesh=_mesh(), scratch_types=scratch,
                     compiler_params=pltpu.CompilerParams(use_tc_tiling_on_sc=False))
    if with_cnt:
        return kern(u, src_rs, dst_rs, z64, z8, ones8)
    return kern(u, src_rs, dst_rs, z64)


def _sc_score(p, q, srcl_rs, dstl_rs):
    def body(p_hbm, q_hbm, srcl_hbm, dstl_hbm, out_hbm,
             p_v, q_v, si_v, di_v, out_v):
        cid = lax.axis_index("c")
        sid = lax.axis_index("s")
        wid = cid * NS + sid
        pltpu.sync_copy(p_hbm, p_v)
        pltpu.sync_copy(q_hbm, q_v)
        pltpu.sync_copy(srcl_hbm.at[wid], si_v)
        pltpu.sync_copy(dstl_hbm.at[wid], di_v)

        def step(t, carry):
            s = pl.ds(t * 16, 16)
            pv = plsc.load_gather(p_v, [si_v[s]])
            qv = plsc.load_gather(q_v, [di_v[s]])
            out_v[s] = pv + qv
            return carry

        lax.fori_loop(0, LT // 16, step, 0)
        pltpu.sync_copy(out_v, out_hbm.at[pl.ds(wid * LT, LT)])

    kern = pl.kernel(
        body,
        out_type=jax.ShapeDtypeStruct((L_PAD,), jnp.float32),
        mesh=_mesh(),
        scratch_types=[pltpu.VMEM((N,), jnp.float32),
                       pltpu.VMEM((N,), jnp.float32),
                       pltpu.VMEM((LT,), jnp.int32),
                       pltpu.VMEM((LT,), jnp.int32),
                       pltpu.VMEM((LT,), jnp.float32)],
        compiler_params=pltpu.CompilerParams(use_tc_tiling_on_sc=False,
                                             needs_layout_passes=False))
    return kern(p, q, srcl_rs, dstl_rs)


_BR = 1000


def _dense1(x, W_l, W_r, b):
    def body(x_ref, wl_ref, wr_ref, b_ref, u_ref, r_ref):
        xb = x_ref[...]
        u_ref[...] = jnp.dot(xb, wl_ref[...], preferred_element_type=jnp.float32)
        r_ref[...] = (jnp.dot(xb, wr_ref[...], preferred_element_type=jnp.float32)
                      + b_ref[...])

    return pl.pallas_call(
        body,
        grid=(N // _BR,),
        in_specs=[pl.BlockSpec((_BR, 128), lambda i: (i, 0)),
                  pl.BlockSpec((128, 64), lambda i: (0, 0)),
                  pl.BlockSpec((128, 64), lambda i: (0, 0)),
                  pl.BlockSpec((1, 64), lambda i: (0, 0))],
        out_specs=[pl.BlockSpec((_BR, 64), lambda i: (i, 0)),
                   pl.BlockSpec((_BR, 64), lambda i: (i, 0))],
        out_shape=[jax.ShapeDtypeStruct((N, 64), jnp.float32),
                   jax.ShapeDtypeStruct((N, 64), jnp.float32)],
    )(x, W_l, W_r, b)


def _dense2(a0, a1, c0, c1, r1, W_l, W_r, b):
    def body(a0_ref, a1_ref, c0_ref, c1_ref, r1_ref, wl_ref, wr_ref, b_ref,
             u_ref, r_ref, ci_ref):
        cnt = c0_ref[...][:, 0:1] + c1_ref[...][:, 0:1]
        ci = 1.0 / jnp.maximum(cnt, 1.0)
        h = jnp.maximum((a0_ref[...] + a1_ref[...]) * ci + r1_ref[...], 0.0)
        u_ref[...] = jnp.dot(h, wl_ref[...], preferred_element_type=jnp.float32)
        r_ref[...] = (jnp.dot(h, wr_ref[...], preferred_element_type=jnp.float32)
                      + b_ref[...])
        ci_ref[...] = ci

    return pl.pallas_call(
        body,
        grid=(N // _BR,),
        in_specs=[pl.BlockSpec((_BR, 64), lambda i: (i, 0)),
                  pl.BlockSpec((_BR, 64), lambda i: (i, 0)),
                  pl.BlockSpec((_BR, CW), lambda i: (i, 0)),
                  pl.BlockSpec((_BR, CW), lambda i: (i, 0)),
                  pl.BlockSpec((_BR, 64), lambda i: (i, 0)),
                  pl.BlockSpec((64, 64), lambda i: (0, 0)),
                  pl.BlockSpec((64, 64), lambda i: (0, 0)),
                  pl.BlockSpec((1, 64), lambda i: (0, 0))],
        out_specs=[pl.BlockSpec((_BR, 64), lambda i: (i, 0)),
                   pl.BlockSpec((_BR, 64), lambda i: (i, 0)),
                   pl.BlockSpec((_BR, 1), lambda i: (i, 0))],
        out_shape=[jax.ShapeDtypeStruct((N, 64), jnp.float32),
                   jax.ShapeDtypeStruct((N, 64), jnp.float32),
                   jax.ShapeDtypeStruct((N, 1), jnp.float32)],
    )(a0, a1, c0, c1, r1, W_l, W_r, b)


def _dense3(a0, a1, ci, r2, Wsr, bsr):
    def body(a0_ref, a1_ref, ci_ref, r2_ref, ws_ref, bs_ref, pq_ref):
        h = jnp.maximum((a0_ref[...] + a1_ref[...]) * ci_ref[...] + r2_ref[...],
                        0.0)
        pq_ref[...] = (jnp.dot(h, ws_ref[...], preferred_element_type=jnp.float32)
                       + bs_ref[...])

    return pl.pallas_call(
        body,
        grid=(N // _BR,),
        in_specs=[pl.BlockSpec((_BR, 64), lambda i: (i, 0)),
                  pl.BlockSpec((_BR, 64), lambda i: (i, 0)),
                  pl.BlockSpec((_BR, 1), lambda i: (i, 0)),
                  pl.BlockSpec((_BR, 64), lambda i: (i, 0)),
                  pl.BlockSpec((64, 2), lambda i: (0, 0)),
                  pl.BlockSpec((1, 2), lambda i: (0, 0))],
        out_specs=pl.BlockSpec((_BR, 2), lambda i: (i, 0)),
        out_shape=jax.ShapeDtypeStruct((N, 2), jnp.float32),
    )(a0, a1, ci, r2, Wsr, bsr)


def kernel(x, edge_index, edge_label_index, W_l1, W_r1, b1, W_l2, W_r2, b2,
           Ws, bs):
    src = edge_index[0].astype(jnp.int32)
    dst = edge_index[1].astype(jnp.int32)
    pad = E_PAD - E
    src_rs = jnp.concatenate([src, jnp.zeros((pad,), jnp.int32)]).reshape(
        NW, J, EB)
    dst_rs = jnp.concatenate([dst, jnp.full((pad,), N, jnp.int32)]).reshape(
        NW, J, EB)
    lpad = L_PAD - LBL
    srcl = edge_label_index[0].astype(jnp.int32)
    dstl = edge_label_index[1].astype(jnp.int32)
    srcl_rs = jnp.concatenate([srcl, jnp.zeros((lpad,), jnp.int32)]).reshape(
        NW, LT)
    dstl_rs = jnp.concatenate([dstl, jnp.zeros((lpad,), jnp.int32)]).reshape(
        NW, LT)

    z64 = jnp.zeros((NPAD, 64), jnp.float32)
    z8 = jnp.zeros((NPAD, CW), jnp.float32)
    ones8 = jnp.ones((EB, CW), jnp.float32)

    u1, r1 = _dense1(x, W_l1, W_r1, b1.reshape(1, 64))
    a0, a1, c0, c1 = _sc_agg(u1, src_rs, dst_rs, z64, z8, ones8, True)
    u2, r2, ci = _dense2(a0[:N], a1[:N], c0[:N], c1[:N], r1, W_l2, W_r2,
                         b2.reshape(1, 64))
    b0, b1_ = _sc_agg(u2, src_rs, dst_rs, z64, None, None, False)
    Wsr = jnp.concatenate([Ws[:64], Ws[64:]], axis=1)
    bsr = jnp.stack([bs[0], jnp.zeros((), jnp.float32)]).reshape(1, 2)
    pq = _dense3(b0[:N], b1_[:N], ci, r2, Wsr, bsr)
    p = pq[:, 0]
    q = pq[:, 1]
    out_pad = _sc_score(p, q, srcl_rs, dstl_rs)
    return out_pad[:LBL]


# speedup vs baseline: 13.8487x; 1.8569x over previous
import functools

import jax
import jax.numpy as jnp
from jax import lax
from jax.experimental import pallas as pl
from jax.experimental.pallas import tpu as pltpu
from jax.experimental.pallas import tpu_sc as plsc

N = 10000
E = 320000
LBL = 100000
NC, NS = 2, 16
NW = NC * NS
EB = 128
J = 80
E_PAD = NW * J * EB
J2 = 160
LT = 3136
L_PAD = NW * LT
NPAD = 10112
RZ = NPAD // NS
CW = 8
NBUF = 4

@functools.cache
def _mesh():
    return plsc.VectorSubcoreMesh(core_axis_name="c", subcore_axis_name="s",
                                  num_cores=NC, num_subcores=NS)


def _sc_agg(u_lo, u_hi, src_rs, dst_rs, z32, z8, ones8, with_cnt):
    out_type = [jax.ShapeDtypeStruct((NPAD, 32), jnp.float32),
                jax.ShapeDtypeStruct((NPAD, 32), jnp.float32)]
    scratch = [pltpu.VMEM((J2, EB), jnp.int32),
               pltpu.VMEM((J2, EB), jnp.int32),
               pltpu.VMEM((NBUF, EB, 32), jnp.float32),
               pltpu.VMEM_SHARED((NPAD, 32), jnp.float32),
               pltpu.VMEM_SHARED((N, 32), jnp.float32),
               *[pltpu.SemaphoreType.DMA] * NBUF]
    if with_cnt:
        out_type += [jax.ShapeDtypeStruct((NPAD, CW), jnp.float32)]
        scratch += [pltpu.VMEM((EB, CW), jnp.float32),
                    pltpu.VMEM_SHARED((NPAD, CW), jnp.float32),
                    *[pltpu.SemaphoreType.DMA] * NBUF]

    def body(*refs):
        if with_cnt:
            (ulo_hbm, uhi_hbm, src_hbm, dst_hbm, z32_hbm, z8_hbm, ones_hbm,
             agg_lo_hbm, agg_hi_hbm, cnt_hbm,
             sidx, didx, rows, acc_sh, u_sh, g0, g1, g2, g3,
             ones_v, cnt_sh, c0, c1, c2, c3) = refs
            csem = [c0, c1, c2, c3]
        else:
            (ulo_hbm, uhi_hbm, src_hbm, dst_hbm, z32_hbm,
             agg_lo_hbm, agg_hi_hbm,
             sidx, didx, rows, acc_sh, u_sh, g0, g1, g2, g3) = refs
        gsem = [g0, g1, g2, g3]
        cid = lax.axis_index("c")
        sid = lax.axis_index("s")

        pltpu.sync_copy(z32_hbm.at[pl.ds(sid * RZ, RZ)],
                        acc_sh.at[pl.ds(sid * RZ, RZ)])
        us = pl.ds(sid * (N // NS), N // NS)

        @pl.when(cid == 0)
        def _():
            pltpu.sync_copy(ulo_hbm.at[us], u_sh.at[us])

        @pl.when(cid == 1)
        def _():
            pltpu.sync_copy(uhi_hbm.at[us], u_sh.at[us])

        pltpu.sync_copy(src_hbm.at[sid], sidx)
        pltpu.sync_copy(dst_hbm.at[sid], didx)
        if with_cnt:
            @pl.when(cid == 0)
            def _():
                pltpu.sync_copy(z8_hbm.at[pl.ds(sid * RZ, RZ)],
                                cnt_sh.at[pl.ds(sid * RZ, RZ)])
                pltpu.sync_copy(ones_hbm, ones_v)
        plsc.subcore_barrier()

        for b in range(NBUF):
            pltpu.async_copy(u_sh.at[sidx.at[b]], rows.at[b], gsem[b])

        def step(jj, carry):
            for b in range(NBUF):
                j = jj * NBUF + b
                pltpu.make_async_copy(u_sh.at[sidx.at[j]], rows.at[b],
                                      gsem[b]).wait()
                pltpu.sync_copy(rows.at[b], acc_sh.at[didx.at[j]], add=True)
                if with_cnt:
                    @pl.when(jnp.logical_and(cid == 0, j >= NBUF))
                    def _():
                        pltpu.make_async_copy(ones_v, cnt_sh.at[didx.at[0]],
                                              csem[b]).wait()

                    @pl.when(cid == 0)
                    def _():
                        pltpu.async_copy(ones_v, cnt_sh.at[didx.at[j]],
                                         csem[b], add=True)

                @pl.when(j + NBUF < J2)
                def _():
                    pltpu.async_copy(u_sh.at[sidx.at[j + NBUF]], rows.at[b],
                                     gsem[b])
            return carry

        lax.fori_loop(0, J2 // NBUF, step, 0)
        if with_cnt:
            @pl.when(cid == 0)
            def _():
                for b in range(NBUF):
                    pltpu.make_async_copy(ones_v, cnt_sh.at[didx.at[0]],
                                          csem[b]).wait()
        plsc.subcore_barrier()

        rs = pl.ds(sid * RZ, RZ)

        @pl.when(cid == 0)
        def _():
            pltpu.sync_copy(acc_sh.at[rs], agg_lo_hbm.at[rs])
            if with_cnt:
                pltpu.sync_copy(cnt_sh.at[rs], cnt_hbm.at[rs])

        @pl.when(cid == 1)
        def _():
            pltpu.sync_copy(acc_sh.at[rs], agg_hi_hbm.at[rs])

    kern = pl.kernel(body, out_type=out_type, mesh=_mesh(), scratch_types=scratch,
                     compiler_params=pltpu.CompilerParams(
                         use_tc_tiling_on_sc=False, needs_layout_passes=False))
    if with_cnt:
        return kern(u_lo, u_hi, src_rs, dst_rs, z32, z8, ones8)
    return kern(u_lo, u_hi, src_rs, dst_rs, z32)


def _sc_score(p, q, srcl_rs, dstl_rs):
    def body(p_hbm, q_hbm, srcl_hbm, dstl_hbm, out_hbm,
             p_v, q_v, si_v, di_v, out_v):
        cid = lax.axis_index("c")
        sid = lax.axis_index("s")
        wid = cid * NS + sid
        pltpu.sync_copy(p_hbm, p_v)
        pltpu.sync_copy(q_hbm, q_v)
        pltpu.sync_copy(srcl_hbm.at[wid], si_v)
        pltpu.sync_copy(dstl_hbm.at[wid], di_v)

        def step(t, carry):
            s = pl.ds(t * 16, 16)
            pv = plsc.load_gather(p_v, [si_v[s]])
            qv = plsc.load_gather(q_v, [di_v[s]])
            out_v[s] = pv + qv
            return carry

        lax.fori_loop(0, LT // 16, step, 0)
        pltpu.sync_copy(out_v, out_hbm.at[pl.ds(wid * LT, LT)])

    kern = pl.kernel(
        body,
        out_type=jax.ShapeDtypeStruct((L_PAD,), jnp.float32),
        mesh=_mesh(),
        scratch_types=[pltpu.VMEM((N,), jnp.float32),
                       pltpu.VMEM((N,), jnp.float32),
                       pltpu.VMEM((LT,), jnp.int32),
                       pltpu.VMEM((LT,), jnp.int32),
                       pltpu.VMEM((LT,), jnp.float32)],
        compiler_params=pltpu.CompilerParams(use_tc_tiling_on_sc=False,
                                             needs_layout_passes=False))
    return kern(p, q, srcl_rs, dstl_rs)


_BR = 1000


def _dense1(x, W_l, W_r, b):
    def body(x_ref, wl_ref, wr_ref, b_ref, u_lo_ref, u_hi_ref, r_ref):
        xb = x_ref[...]
        u = jnp.dot(xb, wl_ref[...], preferred_element_type=jnp.float32)
        u_lo_ref[...] = u[:, :32]
        u_hi_ref[...] = u[:, 32:]
        r_ref[...] = (jnp.dot(xb, wr_ref[...], preferred_element_type=jnp.float32)
                      + b_ref[...])

    return pl.pallas_call(
        body,
        grid=(N // _BR,),
        in_specs=[pl.BlockSpec((_BR, 128), lambda i: (i, 0)),
                  pl.BlockSpec((128, 64), lambda i: (0, 0)),
                  pl.BlockSpec((128, 64), lambda i: (0, 0)),
                  pl.BlockSpec((1, 64), lambda i: (0, 0))],
        out_specs=[pl.BlockSpec((_BR, 32), lambda i: (i, 0)),
                   pl.BlockSpec((_BR, 32), lambda i: (i, 0)),
                   pl.BlockSpec((_BR, 64), lambda i: (i, 0))],
        out_shape=[jax.ShapeDtypeStruct((N, 32), jnp.float32),
                   jax.ShapeDtypeStruct((N, 32), jnp.float32),
                   jax.ShapeDtypeStruct((N, 64), jnp.float32)],
    )(x, W_l, W_r, b)


def _dense2(a0, a1, cparts, r1, W_l, W_r, b):
    def body(a0_ref, a1_ref, c_ref, r1_ref, wl_ref, wr_ref, b_ref,
             u_lo_ref, u_hi_ref, r_ref, ci_ref):
        cnt = c_ref[...][:, 0:1]
        ci = 1.0 / jnp.maximum(cnt, 1.0)
        mean = jnp.concatenate([a0_ref[...], a1_ref[...]], axis=1) * ci
        h = jnp.maximum(mean + r1_ref[...], 0.0)
        u = jnp.dot(h, wl_ref[...], preferred_element_type=jnp.float32)
        u_lo_ref[...] = u[:, :32]
        u_hi_ref[...] = u[:, 32:]
        r_ref[...] = (jnp.dot(h, wr_ref[...], preferred_element_type=jnp.float32)
                      + b_ref[...])
        ci_ref[...] = ci

    return pl.pallas_call(
        body,
        grid=(N // _BR,),
        in_specs=[pl.BlockSpec((_BR, 32), lambda i: (i, 0)),
                  pl.BlockSpec((_BR, 32), lambda i: (i, 0)),
                  pl.BlockSpec((_BR, CW), lambda i: (i, 0)),
                  pl.BlockSpec((_BR, 64), lambda i: (i, 0)),
                  pl.BlockSpec((64, 64), lambda i: (0, 0)),
                  pl.BlockSpec((64, 64), lambda i: (0, 0)),
                  pl.BlockSpec((1, 64), lambda i: (0, 0))],
        out_specs=[pl.BlockSpec((_BR, 32), lambda i: (i, 0)),
                   pl.BlockSpec((_BR, 32), lambda i: (i, 0)),
                   pl.BlockSpec((_BR, 64), lambda i: (i, 0)),
                   pl.BlockSpec((_BR, 1), lambda i: (i, 0))],
        out_shape=[jax.ShapeDtypeStruct((N, 32), jnp.float32),
                   jax.ShapeDtypeStruct((N, 32), jnp.float32),
                   jax.ShapeDtypeStruct((N, 64), jnp.float32),
                   jax.ShapeDtypeStruct((N, 1), jnp.float32)],
    )(a0, a1, cparts, r1, W_l, W_r, b)


def _dense3(a0, a1, ci, r2, Wsr, bsr):
    def body(a0_ref, a1_ref, ci_ref, r2_ref, ws_ref, bs_ref, pq_ref):
        mean = jnp.concatenate([a0_ref[...], a1_ref[...]], axis=1) * ci_ref[...]
        h = jnp.maximum(mean + r2_ref[...], 0.0)
        pq_ref[...] = (jnp.dot(h, ws_ref[...], preferred_element_type=jnp.float32)
                       + bs_ref[...])

    return pl.pallas_call(
        body,
        grid=(N // _BR,),
        in_specs=[pl.BlockSpec((_BR, 32), lambda i: (i, 0)),
                  pl.BlockSpec((_BR, 32), lambda i: (i, 0)),
                  pl.BlockSpec((_BR, 1), lambda i: (i, 0)),
                  pl.BlockSpec((_BR, 64), lambda i: (i, 0)),
                  pl.BlockSpec((64, 2), lambda i: (0, 0)),
                  pl.BlockSpec((1, 2), lambda i: (0, 0))],
        out_specs=pl.BlockSpec((_BR, 2), lambda i: (i, 0)),
        out_shape=jax.ShapeDtypeStruct((N, 2), jnp.float32),
    )(a0, a1, ci, r2, Wsr, bsr)


def kernel(x, edge_index, edge_label_index, W_l1, W_r1, b1, W_l2, W_r2, b2,
           Ws, bs):
    src = edge_index[0].astype(jnp.int32)
    dst = edge_index[1].astype(jnp.int32)
    pad = E_PAD - E
    src_rs = jnp.concatenate([src, jnp.zeros((pad,), jnp.int32)]).reshape(
        NS, J2, EB)
    dst_rs = jnp.concatenate([dst, jnp.full((pad,), N, jnp.int32)]).reshape(
        NS, J2, EB)
    lpad = L_PAD - LBL
    srcl = edge_label_index[0].astype(jnp.int32)
    dstl = edge_label_index[1].astype(jnp.int32)
    srcl_rs = jnp.concatenate([srcl, jnp.zeros((lpad,), jnp.int32)]).reshape(
        NW, LT)
    dstl_rs = jnp.concatenate([dstl, jnp.zeros((lpad,), jnp.int32)]).reshape(
        NW, LT)

    z32 = jnp.zeros((NPAD, 32), jnp.float32)
    z8 = jnp.zeros((NPAD, CW), jnp.float32)
    ones8 = jnp.ones((EB, CW), jnp.float32)

    u1lo, u1hi, r1 = _dense1(x, W_l1, W_r1, b1.reshape(1, 64))
    a0, a1, cnt = _sc_agg(u1lo, u1hi, src_rs, dst_rs, z32, z8, ones8, True)
    u2lo, u2hi, r2, ci = _dense2(a0[:N], a1[:N], cnt[:N], r1, W_l2, W_r2,
                                 b2.reshape(1, 64))
    b0, b1_ = _sc_agg(u2lo, u2hi, src_rs, dst_rs, z32, None, None, False)
    Wsr = jnp.concatenate([Ws[:64], Ws[64:]], axis=1)
    bsr = jnp.stack([bs[0], jnp.zeros((), jnp.float32)]).reshape(1, 2)
    pq = _dense3(b0[:N], b1_[:N], ci, r2, Wsr, bsr)
    p = pq[:, 0]
    q = pq[:, 1]
    out_pad = _sc_score(p, q, srcl_rs, dstl_rs)
    return out_pad[:LBL]


# speedup vs baseline: 14.1184x; 1.0195x over previous
import functools

import jax
import jax.numpy as jnp
from jax import lax
from jax.experimental import pallas as pl
from jax.experimental.pallas import tpu as pltpu
from jax.experimental.pallas import tpu_sc as plsc

N = 10000
E = 320000
LBL = 100000
NC, NS = 2, 16
NW = NC * NS
EB = 128
J = 80
E_PAD = NW * J * EB
J2 = 160
LT = 3136
L_PAD = NW * LT
NPAD = 10112
RZ = NPAD // NS
CW = 8
NBUF = 4

@functools.cache
def _mesh():
    return plsc.VectorSubcoreMesh(core_axis_name="c", subcore_axis_name="s",
                                  num_cores=NC, num_subcores=NS)


def _sc_agg(u_lo, u_hi, src_rs, dst_rs, z32, z8, ones8, with_cnt):
    out_type = [jax.ShapeDtypeStruct((NPAD, 32), jnp.float32),
                jax.ShapeDtypeStruct((NPAD, 32), jnp.float32)]
    scratch = [pltpu.VMEM((J2, EB), jnp.int32),
               pltpu.VMEM((J2, EB), jnp.int32),
               pltpu.VMEM((NBUF, EB, 32), jnp.float32),
               pltpu.VMEM_SHARED((NPAD, 32), jnp.float32),
               pltpu.VMEM_SHARED((N, 32), jnp.float32),
               *[pltpu.SemaphoreType.DMA] * NBUF]
    if with_cnt:
        out_type += [jax.ShapeDtypeStruct((NPAD, CW), jnp.float32)]
        scratch += [pltpu.VMEM((EB, CW), jnp.float32),
                    pltpu.VMEM_SHARED((NPAD, CW), jnp.float32),
                    *[pltpu.SemaphoreType.DMA] * NBUF]

    def body(*refs):
        if with_cnt:
            (ulo_hbm, uhi_hbm, src_hbm, dst_hbm, z32_hbm, z8_hbm, ones_hbm,
             agg_lo_hbm, agg_hi_hbm, cnt_hbm,
             sidx, didx, rows, acc_sh, u_sh, g0, g1, g2, g3,
             ones_v, cnt_sh, c0, c1, c2, c3) = refs
            csem = [c0, c1, c2, c3]
        else:
            (ulo_hbm, uhi_hbm, src_hbm, dst_hbm, z32_hbm,
             agg_lo_hbm, agg_hi_hbm,
             sidx, didx, rows, acc_sh, u_sh, g0, g1, g2, g3) = refs
        gsem = [g0, g1, g2, g3]
        cid = lax.axis_index("c")
        sid = lax.axis_index("s")

        pltpu.sync_copy(z32_hbm.at[pl.ds(sid * RZ, RZ)],
                        acc_sh.at[pl.ds(sid * RZ, RZ)])
        us = pl.ds(sid * (N // NS), N // NS)

        @pl.when(cid == 0)
        def _():
            pltpu.sync_copy(ulo_hbm.at[us], u_sh.at[us])

        @pl.when(cid == 1)
        def _():
            pltpu.sync_copy(uhi_hbm.at[us], u_sh.at[us])

        pltpu.sync_copy(src_hbm.at[sid], sidx)
        pltpu.sync_copy(dst_hbm.at[sid], didx)
        if with_cnt:
            @pl.when(cid == 0)
            def _():
                pltpu.sync_copy(z8_hbm.at[pl.ds(sid * RZ, RZ)],
                                cnt_sh.at[pl.ds(sid * RZ, RZ)])
                pltpu.sync_copy(ones_hbm, ones_v)
        plsc.subcore_barrier()

        for b in range(NBUF):
            pltpu.async_copy(u_sh.at[sidx.at[b]], rows.at[b], gsem[b])

        def step(jj, carry):
            for b in range(NBUF):
                j = jj * NBUF + b
                pltpu.make_async_copy(u_sh.at[sidx.at[j]], rows.at[b],
                                      gsem[b]).wait()
                pltpu.sync_copy(rows.at[b], acc_sh.at[didx.at[j]], add=True)
                if with_cnt:
                    @pl.when(jnp.logical_and(cid == 0, j >= NBUF))
                    def _():
                        pltpu.make_async_copy(ones_v, cnt_sh.at[didx.at[0]],
                                              csem[b]).wait()

                    @pl.when(cid == 0)
                    def _():
                        pltpu.async_copy(ones_v, cnt_sh.at[didx.at[j]],
                                         csem[b], add=True)

                @pl.when(j + NBUF < J2)
                def _():
                    pltpu.async_copy(u_sh.at[sidx.at[j + NBUF]], rows.at[b],
                                     gsem[b])
            return carry

        lax.fori_loop(0, J2 // NBUF, step, 0)
        if with_cnt:
            @pl.when(cid == 0)
            def _():
                for b in range(NBUF):
                    pltpu.make_async_copy(ones_v, cnt_sh.at[didx.at[0]],
                                          csem[b]).wait()
        plsc.subcore_barrier()

        rs = pl.ds(sid * RZ, RZ)

        @pl.when(cid == 0)
        def _():
            pltpu.sync_copy(acc_sh.at[rs], agg_lo_hbm.at[rs])
            if with_cnt:
                pltpu.sync_copy(cnt_sh.at[rs], cnt_hbm.at[rs])

        @pl.when(cid == 1)
        def _():
            pltpu.sync_copy(acc_sh.at[rs], agg_hi_hbm.at[rs])

    kern = pl.kernel(body, out_type=out_type, mesh=_mesh(), scratch_types=scratch,
                     compiler_params=pltpu.CompilerParams(
                         use_tc_tiling_on_sc=False, needs_layout_passes=False))
    if with_cnt:
        return kern(u_lo, u_hi, src_rs, dst_rs, z32, z8, ones8)
    return kern(u_lo, u_hi, src_rs, dst_rs, z32)


def _sc_score(pq, srcl_rs, dstl_rs):
    def body(pq_hbm, srcl_hbm, dstl_hbm, out_hbm,
             pq_v, si_v, di_v, out_v):
        cid = lax.axis_index("c")
        sid = lax.axis_index("s")
        wid = cid * NS + sid
        pltpu.sync_copy(pq_hbm, pq_v)
        pltpu.sync_copy(srcl_hbm.at[wid], si_v)
        pltpu.sync_copy(dstl_hbm.at[wid], di_v)
        col0 = jnp.zeros((16,), jnp.int32)
        col1 = jnp.ones((16,), jnp.int32)

        def step(t, carry):
            sl = pl.ds(t * 16, 16)
            pv = plsc.load_gather(pq_v, [si_v[sl], col0])
            qv = plsc.load_gather(pq_v, [di_v[sl], col1])
            out_v[sl] = pv + qv
            return carry

        lax.fori_loop(0, LT // 16, step, 0)
        pltpu.sync_copy(out_v, out_hbm.at[pl.ds(wid * LT, LT)])

    kern = pl.kernel(
        body,
        out_type=jax.ShapeDtypeStruct((L_PAD,), jnp.float32),
        mesh=_mesh(),
        scratch_types=[pltpu.VMEM((N, 2), jnp.float32),
                       pltpu.VMEM((LT,), jnp.int32),
                       pltpu.VMEM((LT,), jnp.int32),
                       pltpu.VMEM((LT,), jnp.float32)],
        compiler_params=pltpu.CompilerParams(use_tc_tiling_on_sc=False,
                                             needs_layout_passes=False))
    return kern(pq, srcl_rs, dstl_rs)


_BR = 1000


def _dense1(x, W_l, W_r, b):
    def body(x_ref, wl_ref, wr_ref, b_ref, u_lo_ref, u_hi_ref, r_ref):
        xb = x_ref[...]
        u = jnp.dot(xb, wl_ref[...], preferred_element_type=jnp.float32)
        u_lo_ref[...] = u[:, :32]
        u_hi_ref[...] = u[:, 32:]
        r_ref[...] = (jnp.dot(xb, wr_ref[...], preferred_element_type=jnp.float32)
                      + b_ref[...])

    return pl.pallas_call(
        body,
        grid=(N // _BR,),
        in_specs=[pl.BlockSpec((_BR, 128), lambda i: (i, 0)),
                  pl.BlockSpec((128, 64), lambda i: (0, 0)),
                  pl.BlockSpec((128, 64), lambda i: (0, 0)),
                  pl.BlockSpec((1, 64), lambda i: (0, 0))],
        out_specs=[pl.BlockSpec((_BR, 32), lambda i: (i, 0)),
                   pl.BlockSpec((_BR, 32), lambda i: (i, 0)),
                   pl.BlockSpec((_BR, 64), lambda i: (i, 0))],
        out_shape=[jax.ShapeDtypeStruct((N, 32), jnp.float32),
                   jax.ShapeDtypeStruct((N, 32), jnp.float32),
                   jax.ShapeDtypeStruct((N, 64), jnp.float32)],
    )(x, W_l, W_r, b)


def _dense2(a0, a1, cparts, r1, W_l, W_r, b):
    def body(a0_ref, a1_ref, c_ref, r1_ref, wl_ref, wr_ref, b_ref,
             u_lo_ref, u_hi_ref, r_ref, ci_ref):
        cnt = c_ref[...][:, 0:1]
        ci = 1.0 / jnp.maximum(cnt, 1.0)
        mean = jnp.concatenate([a0_ref[...], a1_ref[...]], axis=1) * ci
        h = jnp.maximum(mean + r1_ref[...], 0.0)
        u = jnp.dot(h, wl_ref[...], preferred_element_type=jnp.float32)
        u_lo_ref[...] = u[:, :32]
        u_hi_ref[...] = u[:, 32:]
        r_ref[...] = (jnp.dot(h, wr_ref[...], preferred_element_type=jnp.float32)
                      + b_ref[...])
        ci_ref[...] = ci

    return pl.pallas_call(
        body,
        grid=(N // _BR,),
        in_specs=[pl.BlockSpec((_BR, 32), lambda i: (i, 0)),
                  pl.BlockSpec((_BR, 32), lambda i: (i, 0)),
                  pl.BlockSpec((_BR, CW), lambda i: (i, 0)),
                  pl.BlockSpec((_BR, 64), lambda i: (i, 0)),
                  pl.BlockSpec((64, 64), lambda i: (0, 0)),
                  pl.BlockSpec((64, 64), lambda i: (0, 0)),
                  pl.BlockSpec((1, 64), lambda i: (0, 0))],
        out_specs=[pl.BlockSpec((_BR, 32), lambda i: (i, 0)),
                   pl.BlockSpec((_BR, 32), lambda i: (i, 0)),
                   pl.BlockSpec((_BR, 64), lambda i: (i, 0)),
                   pl.BlockSpec((_BR, 1), lambda i: (i, 0))],
        out_shape=[jax.ShapeDtypeStruct((N, 32), jnp.float32),
                   jax.ShapeDtypeStruct((N, 32), jnp.float32),
                   jax.ShapeDtypeStruct((N, 64), jnp.float32),
                   jax.ShapeDtypeStruct((N, 1), jnp.float32)],
    )(a0, a1, cparts, r1, W_l, W_r, b)


def _dense3(a0, a1, ci, r2, Wsr, bsr):
    def body(a0_ref, a1_ref, ci_ref, r2_ref, ws_ref, bs_ref, pq_ref):
        mean = jnp.concatenate([a0_ref[...], a1_ref[...]], axis=1) * ci_ref[...]
        h = jnp.maximum(mean + r2_ref[...], 0.0)
        pq_ref[...] = (jnp.dot(h, ws_ref[...], preferred_element_type=jnp.float32)
                       + bs_ref[...])

    return pl.pallas_call(
        body,
        grid=(N // _BR,),
        in_specs=[pl.BlockSpec((_BR, 32), lambda i: (i, 0)),
                  pl.BlockSpec((_BR, 32), lambda i: (i, 0)),
                  pl.BlockSpec((_BR, 1), lambda i: (i, 0)),
                  pl.BlockSpec((_BR, 64), lambda i: (i, 0)),
                  pl.BlockSpec((64, 2), lambda i: (0, 0)),
                  pl.BlockSpec((1, 2), lambda i: (0, 0))],
        out_specs=pl.BlockSpec((_BR, 2), lambda i: (i, 0)),
        out_shape=jax.ShapeDtypeStruct((N, 2), jnp.float32),
    )(a0, a1, ci, r2, Wsr, bsr)


def kernel(x, edge_index, edge_label_index, W_l1, W_r1, b1, W_l2, W_r2, b2,
           Ws, bs):
    src = edge_index[0].astype(jnp.int32)
    dst = edge_index[1].astype(jnp.int32)
    pad = E_PAD - E
    src_rs = jnp.concatenate([src, jnp.zeros((pad,), jnp.int32)]).reshape(
        NS, J2, EB)
    dst_rs = jnp.concatenate([dst, jnp.full((pad,), N, jnp.int32)]).reshape(
        NS, J2, EB)
    lpad = L_PAD - LBL
    srcl = edge_label_index[0].astype(jnp.int32)
    dstl = edge_label_index[1].astype(jnp.int32)
    srcl_rs = jnp.concatenate([srcl, jnp.zeros((lpad,), jnp.int32)]).reshape(
        NW, LT)
    dstl_rs = jnp.concatenate([dstl, jnp.zeros((lpad,), jnp.int32)]).reshape(
        NW, LT)

    z32 = jnp.zeros((NPAD, 32), jnp.float32)
    z8 = jnp.zeros((NPAD, CW), jnp.float32)
    ones8 = jnp.ones((EB, CW), jnp.float32)

    u1lo, u1hi, r1 = _dense1(x, W_l1, W_r1, b1.reshape(1, 64))
    a0, a1, cnt = _sc_agg(u1lo, u1hi, src_rs, dst_rs, z32, z8, ones8, True)
    u2lo, u2hi, r2, ci = _dense2(a0, a1, cnt, r1, W_l2, W_r2,
                                 b2.reshape(1, 64))
    b0, b1_ = _sc_agg(u2lo, u2hi, src_rs, dst_rs, z32, None, None, False)
    Wsr = jnp.concatenate([Ws[:64], Ws[64:]], axis=1)
    bsr = jnp.stack([bs[0], jnp.zeros((), jnp.float32)]).reshape(1, 2)
    pq = _dense3(b0, b1_, ci, r2, Wsr, bsr)
    out_pad = _sc_score(pq, srcl_rs, dstl_rs)
    return out_pad[:LBL]


# speedup vs baseline: 14.3979x; 1.0198x over previous
import functools

import jax
import jax.numpy as jnp
from jax import lax
from jax.experimental import pallas as pl
from jax.experimental.pallas import tpu as pltpu
from jax.experimental.pallas import tpu_sc as plsc

N = 10000
E = 320000
LBL = 100000
NC, NS = 2, 16
NW = NC * NS
EB = 128
J = 80
E_PAD = NW * J * EB
J2 = 160
LT = 3136
L_PAD = NW * LT
NPAD = 10112
RZ = NPAD // NS
CW = 8
NBUF = 4

@functools.cache
def _mesh():
    return plsc.VectorSubcoreMesh(core_axis_name="c", subcore_axis_name="s",
                                  num_cores=NC, num_subcores=NS)


NB = 2500
NBT = 156
CNT_SPLIT = NBT // 2


def _sc_agg(u_lo, u_hi, src_b, dst_b, z32, z8, ones8, with_cnt):
    out_type = [jax.ShapeDtypeStruct((NPAD, 32), jnp.float32),
                jax.ShapeDtypeStruct((NPAD, 32), jnp.float32)]
    scratch = [pltpu.VMEM((NBT + 1, EB), jnp.int32),
               pltpu.VMEM((NBT + 1, EB), jnp.int32),
               pltpu.VMEM((NBUF, EB, 32), jnp.float32),
               pltpu.VMEM_SHARED((NPAD, 32), jnp.float32),
               pltpu.VMEM_SHARED((N, 32), jnp.float32),
               *[pltpu.SemaphoreType.DMA] * NBUF]
    if with_cnt:
        out_type += [jax.ShapeDtypeStruct((NPAD, CW), jnp.float32),
                     jax.ShapeDtypeStruct((NPAD, CW), jnp.float32)]
        scratch += [pltpu.VMEM((EB, CW), jnp.float32),
                    pltpu.VMEM_SHARED((NPAD, CW), jnp.float32)]

    def body(*refs):
        if with_cnt:
            (ulo_hbm, uhi_hbm, src_hbm, dst_hbm, z32_hbm, z8_hbm, ones_hbm,
             agg_lo_hbm, agg_hi_hbm, cnt0_hbm, cnt1_hbm,
             sidx, didx, rows, acc_sh, u_sh, g0, g1, g2, g3,
             ones_v, cnt_sh) = refs
        else:
            (ulo_hbm, uhi_hbm, src_hbm, dst_hbm, z32_hbm,
             agg_lo_hbm, agg_hi_hbm,
             sidx, didx, rows, acc_sh, u_sh, g0, g1, g2, g3) = refs
        gsem = [g0, g1, g2, g3]
        cid = lax.axis_index("c")
        sid = lax.axis_index("s")
        has_tail = sid < NB - NS * NBT

        pltpu.sync_copy(z32_hbm.at[pl.ds(sid * RZ, RZ)],
                        acc_sh.at[pl.ds(sid * RZ, RZ)])
        us = pl.ds(sid * (N // NS), N // NS)

        @pl.when(cid == 0)
        def _():
            pltpu.sync_copy(ulo_hbm.at[us], u_sh.at[us])

        @pl.when(cid == 1)
        def _():
            pltpu.sync_copy(uhi_hbm.at[us], u_sh.at[us])

        bs_ = pl.ds(sid * NBT, NBT)
        pltpu.sync_copy(src_hbm.at[bs_], sidx.at[pl.ds(0, NBT)])
        pltpu.sync_copy(dst_hbm.at[bs_], didx.at[pl.ds(0, NBT)])

        @pl.when(has_tail)
        def _():
            ts_ = pl.ds(NS * NBT + sid, 1)
            pltpu.sync_copy(src_hbm.at[ts_], sidx.at[pl.ds(NBT, 1)])
            pltpu.sync_copy(dst_hbm.at[ts_], didx.at[pl.ds(NBT, 1)])

        if with_cnt:
            pltpu.sync_copy(z8_hbm.at[pl.ds(sid * RZ, RZ)],
                            cnt_sh.at[pl.ds(sid * RZ, RZ)])
            pltpu.sync_copy(ones_hbm, ones_v)
        plsc.subcore_barrier()

        for b in range(NBUF):
            pltpu.async_copy(u_sh.at[sidx.at[b]], rows.at[b], gsem[b])

        def step(jj, carry):
            for b in range(NBUF):
                j = jj * NBUF + b
                pltpu.make_async_copy(u_sh.at[sidx.at[j]], rows.at[b],
                                      gsem[b]).wait()
                pltpu.sync_copy(rows.at[b], acc_sh.at[didx.at[j]], add=True)
                if with_cnt:
                    mine = lax.select(cid == 0, j < CNT_SPLIT, j >= CNT_SPLIT)

                    @pl.when(mine)
                    def _():
                        pltpu.sync_copy(ones_v, cnt_sh.at[didx.at[j]],
                                        add=True)

                @pl.when(j + NBUF < NBT)
                def _():
                    pltpu.async_copy(u_sh.at[sidx.at[j + NBUF]], rows.at[b],
                                     gsem[b])
            return carry

        lax.fori_loop(0, NBT // NBUF, step, 0)

        @pl.when(has_tail)
        def _():
            pltpu.sync_copy(u_sh.at[sidx.at[NBT]], rows.at[0])
            pltpu.sync_copy(rows.at[0], acc_sh.at[didx.at[NBT]], add=True)
            if with_cnt:
                @pl.when(cid == 1)
                def _():
                    pltpu.sync_copy(ones_v, cnt_sh.at[didx.at[NBT]],
                                    add=True)

        plsc.subcore_barrier()

        rs = pl.ds(sid * RZ, RZ)

        @pl.when(cid == 0)
        def _():
            pltpu.sync_copy(acc_sh.at[rs], agg_lo_hbm.at[rs])
            if with_cnt:
                pltpu.sync_copy(cnt_sh.at[rs], cnt0_hbm.at[rs])

        @pl.when(cid == 1)
        def _():
            pltpu.sync_copy(acc_sh.at[rs], agg_hi_hbm.at[rs])
            if with_cnt:
                pltpu.sync_copy(cnt_sh.at[rs], cnt1_hbm.at[rs])

    kern = pl.kernel(body, out_type=out_type, mesh=_mesh(), scratch_types=scratch,
                     compiler_params=pltpu.CompilerParams(
                         use_tc_tiling_on_sc=False, needs_layout_passes=False))
    if with_cnt:
        return kern(u_lo, u_hi, src_b, dst_b, z32, z8, ones8)
    return kern(u_lo, u_hi, src_b, dst_b, z32)


def _sc_score(pq, srcl, dstl):
    def body(pq_hbm, srcl_hbm, dstl_hbm, out_hbm,
             pq_v, si_v, di_v, out_v):
        cid = lax.axis_index("c")
        sid = lax.axis_index("s")
        wid = cid * NS + sid
        base = jnp.minimum(wid * LT, LBL - LT)
        pltpu.sync_copy(pq_hbm, pq_v)
        pltpu.sync_copy(srcl_hbm.at[pl.ds(base, LT)], si_v)
        pltpu.sync_copy(dstl_hbm.at[pl.ds(base, LT)], di_v)
        col0 = jnp.zeros((16,), jnp.int32)
        col1 = jnp.ones((16,), jnp.int32)

        def step(t, carry):
            sl = pl.ds(t * 16, 16)
            pv = plsc.load_gather(pq_v, [si_v[sl], col0])
            qv = plsc.load_gather(pq_v, [di_v[sl], col1])
            out_v[sl] = pv + qv
            return carry

        lax.fori_loop(0, LT // 16, step, 0)
        pltpu.sync_copy(out_v, out_hbm.at[pl.ds(base, LT)])

    kern = pl.kernel(
        body,
        out_type=jax.ShapeDtypeStruct((LBL,), jnp.float32),
        mesh=_mesh(),
        scratch_types=[pltpu.VMEM((N, 2), jnp.float32),
                       pltpu.VMEM((LT,), jnp.int32),
                       pltpu.VMEM((LT,), jnp.int32),
                       pltpu.VMEM((LT,), jnp.float32)],
        compiler_params=pltpu.CompilerParams(use_tc_tiling_on_sc=False,
                                             needs_layout_passes=False))
    return kern(pq, srcl, dstl)


_BR = 1000


def _dense1(x, W_l, W_r, b):
    def body(x_ref, wl_ref, wr_ref, b_ref, u_lo_ref, u_hi_ref, r_ref):
        xb = x_ref[...]
        u = jnp.dot(xb, wl_ref[...], preferred_element_type=jnp.float32)
        u_lo_ref[...] = u[:, :32]
        u_hi_ref[...] = u[:, 32:]
        r_ref[...] = (jnp.dot(xb, wr_ref[...], preferred_element_type=jnp.float32)
                      + b_ref[...])

    return pl.pallas_call(
        body,
        grid=(N // _BR,),
        in_specs=[pl.BlockSpec((_BR, 128), lambda i: (i, 0)),
                  pl.BlockSpec((128, 64), lambda i: (0, 0)),
                  pl.BlockSpec((128, 64), lambda i: (0, 0)),
                  pl.BlockSpec((1, 64), lambda i: (0, 0))],
        out_specs=[pl.BlockSpec((_BR, 32), lambda i: (i, 0)),
                   pl.BlockSpec((_BR, 32), lambda i: (i, 0)),
                   pl.BlockSpec((_BR, 64), lambda i: (i, 0))],
        out_shape=[jax.ShapeDtypeStruct((N, 32), jnp.float32),
                   jax.ShapeDtypeStruct((N, 32), jnp.float32),
                   jax.ShapeDtypeStruct((N, 64), jnp.float32)],
    )(x, W_l, W_r, b)


def _dense2(a0, a1, c0, c1, r1, W_l, W_r, b):
    def body(a0_ref, a1_ref, c0_ref, c1_ref, r1_ref, wl_ref, wr_ref, b_ref,
             u_lo_ref, u_hi_ref, r_ref, ci_ref):
        cnt = c0_ref[...][:, 0:1] + c1_ref[...][:, 0:1]
        ci = 1.0 / jnp.maximum(cnt, 1.0)
        mean = jnp.concatenate([a0_ref[...], a1_ref[...]], axis=1) * ci
        h = jnp.maximum(mean + r1_ref[...], 0.0)
        u = jnp.dot(h, wl_ref[...], preferred_element_type=jnp.float32)
        u_lo_ref[...] = u[:, :32]
        u_hi_ref[...] = u[:, 32:]
        r_ref[...] = (jnp.dot(h, wr_ref[...], preferred_element_type=jnp.float32)
                      + b_ref[...])
        ci_ref[...] = ci

    return pl.pallas_call(
        body,
        grid=(N // _BR,),
        in_specs=[pl.BlockSpec((_BR, 32), lambda i: (i, 0)),
                  pl.BlockSpec((_BR, 32), lambda i: (i, 0)),
                  pl.BlockSpec((_BR, CW), lambda i: (i, 0)),
                  pl.BlockSpec((_BR, CW), lambda i: (i, 0)),
                  pl.BlockSpec((_BR, 64), lambda i: (i, 0)),
                  pl.BlockSpec((64, 64), lambda i: (0, 0)),
                  pl.BlockSpec((64, 64), lambda i: (0, 0)),
                  pl.BlockSpec((1, 64), lambda i: (0, 0))],
        out_specs=[pl.BlockSpec((_BR, 32), lambda i: (i, 0)),
                   pl.BlockSpec((_BR, 32), lambda i: (i, 0)),
                   pl.BlockSpec((_BR, 64), lambda i: (i, 0)),
                   pl.BlockSpec((_BR, 1), lambda i: (i, 0))],
        out_shape=[jax.ShapeDtypeStruct((N, 32), jnp.float32),
                   jax.ShapeDtypeStruct((N, 32), jnp.float32),
                   jax.ShapeDtypeStruct((N, 64), jnp.float32),
                   jax.ShapeDtypeStruct((N, 1), jnp.float32)],
    )(a0, a1, c0, c1, r1, W_l, W_r, b)


def _dense3(a0, a1, ci, r2, Wsr, bsr):
    def body(a0_ref, a1_ref, ci_ref, r2_ref, ws_ref, bs_ref, pq_ref):
        mean = jnp.concatenate([a0_ref[...], a1_ref[...]], axis=1) * ci_ref[...]
        h = jnp.maximum(mean + r2_ref[...], 0.0)
        pq_ref[...] = (jnp.dot(h, ws_ref[...], preferred_element_type=jnp.float32)
                       + bs_ref[...])

    return pl.pallas_call(
        body,
        grid=(N // _BR,),
        in_specs=[pl.BlockSpec((_BR, 32), lambda i: (i, 0)),
                  pl.BlockSpec((_BR, 32), lambda i: (i, 0)),
                  pl.BlockSpec((_BR, 1), lambda i: (i, 0)),
                  pl.BlockSpec((_BR, 64), lambda i: (i, 0)),
                  pl.BlockSpec((64, 2), lambda i: (0, 0)),
                  pl.BlockSpec((1, 2), lambda i: (0, 0))],
        out_specs=pl.BlockSpec((_BR, 2), lambda i: (i, 0)),
        out_shape=jax.ShapeDtypeStruct((N, 2), jnp.float32),
    )(a0, a1, ci, r2, Wsr, bsr)


def kernel(x, edge_index, edge_label_index, W_l1, W_r1, b1, W_l2, W_r2, b2,
           Ws, bs):
    src_b = edge_index[0].astype(jnp.int32).reshape(NB, EB)
    dst_b = edge_index[1].astype(jnp.int32).reshape(NB, EB)
    srcl = edge_label_index[0].astype(jnp.int32)
    dstl = edge_label_index[1].astype(jnp.int32)

    z32 = jnp.zeros((NPAD, 32), jnp.float32)
    z8 = jnp.zeros((NPAD, CW), jnp.float32)
    ones8 = jnp.ones((EB, CW), jnp.float32)

    u1lo, u1hi, r1 = _dense1(x, W_l1, W_r1, b1.reshape(1, 64))
    a0, a1, c0, c1 = _sc_agg(u1lo, u1hi, src_b, dst_b, z32, z8, ones8, True)
    u2lo, u2hi, r2, ci = _dense2(a0, a1, c0, c1, r1, W_l2, W_r2,
                                 b2.reshape(1, 64))
    b0, b1_ = _sc_agg(u2lo, u2hi, src_b, dst_b, z32, None, None, False)
    Wsr = jnp.concatenate([Ws[:64], Ws[64:]], axis=1)
    bsr = jnp.stack([bs[0], jnp.zeros((), jnp.float32)]).reshape(1, 2)
    pq = _dense3(b0, b1_, ci, r2, Wsr, bsr)
    return _sc_score(pq, srcl, dstl)


# speedup vs baseline: 15.3257x; 1.0644x over previous
import functools

import jax
import jax.numpy as jnp
from jax import lax
from jax.experimental import pallas as pl
from jax.experimental.pallas import tpu as pltpu
from jax.experimental.pallas import tpu_sc as plsc

N = 10000
E = 320000
LBL = 100000
NC, NS = 2, 16
NW = NC * NS
EB = 128
J = 80
E_PAD = NW * J * EB
J2 = 160
LT = 3136
L_PAD = NW * LT
NPAD = 10112
RZ = NPAD // NS
CW = 8
NBUF = 4

@functools.cache
def _mesh():
    return plsc.VectorSubcoreMesh(core_axis_name="c", subcore_axis_name="s",
                                  num_cores=NC, num_subcores=NS)


NB = 2500
NBT = 156
CNT_SPLIT = NBT // 2
NSLOT = 6
LEAD = 3


def _sc_agg(u_lo, u_hi, src_b, dst_b, z32, z8, ones8, with_cnt):
    out_type = [jax.ShapeDtypeStruct((NPAD, 32), jnp.float32),
                jax.ShapeDtypeStruct((NPAD, 32), jnp.float32)]
    scratch = [pltpu.VMEM((NBT + 1, EB), jnp.int32),
               pltpu.VMEM((NBT + 1, EB), jnp.int32),
               pltpu.VMEM((NSLOT, EB, 32), jnp.float32),
               pltpu.VMEM_SHARED((NPAD, 32), jnp.float32),
               pltpu.VMEM_SHARED((N, 32), jnp.float32),
               *[pltpu.SemaphoreType.DMA] * (2 * NSLOT)]
    if with_cnt:
        out_type += [jax.ShapeDtypeStruct((NPAD, CW), jnp.float32),
                     jax.ShapeDtypeStruct((NPAD, CW), jnp.float32)]
        scratch += [pltpu.VMEM((EB, CW), jnp.float32),
                    pltpu.VMEM_SHARED((NPAD, CW), jnp.float32)]

    def body(*refs):
        if with_cnt:
            (ulo_hbm, uhi_hbm, src_hbm, dst_hbm, z32_hbm, z8_hbm, ones_hbm,
             agg_lo_hbm, agg_hi_hbm, cnt0_hbm, cnt1_hbm,
             sidx, didx, rows, acc_sh, u_sh, *sems) = refs[:16 + 2 * NSLOT]
            ones_v, cnt_sh = refs[16 + 2 * NSLOT:]
        else:
            (ulo_hbm, uhi_hbm, src_hbm, dst_hbm, z32_hbm,
             agg_lo_hbm, agg_hi_hbm,
             sidx, didx, rows, acc_sh, u_sh, *sems) = refs
        gsem = list(sems[:NSLOT])
        rsem = list(sems[NSLOT:2 * NSLOT])
        cid = lax.axis_index("c")
        sid = lax.axis_index("s")
        has_tail = sid < NB - NS * NBT

        pltpu.sync_copy(z32_hbm.at[pl.ds(sid * RZ, RZ)],
                        acc_sh.at[pl.ds(sid * RZ, RZ)])
        us = pl.ds(sid * (N // NS), N // NS)

        @pl.when(cid == 0)
        def _():
            pltpu.sync_copy(ulo_hbm.at[us], u_sh.at[us])

        @pl.when(cid == 1)
        def _():
            pltpu.sync_copy(uhi_hbm.at[us], u_sh.at[us])

        bs_ = pl.ds(sid * NBT, NBT)
        pltpu.sync_copy(src_hbm.at[bs_], sidx.at[pl.ds(0, NBT)])
        pltpu.sync_copy(dst_hbm.at[bs_], didx.at[pl.ds(0, NBT)])

        @pl.when(has_tail)
        def _():
            ts_ = pl.ds(NS * NBT + sid, 1)
            pltpu.sync_copy(src_hbm.at[ts_], sidx.at[pl.ds(NBT, 1)])
            pltpu.sync_copy(dst_hbm.at[ts_], didx.at[pl.ds(NBT, 1)])

        if with_cnt:
            pltpu.sync_copy(z8_hbm.at[pl.ds(sid * RZ, RZ)],
                            cnt_sh.at[pl.ds(sid * RZ, RZ)])
            pltpu.sync_copy(ones_hbm, ones_v)
        plsc.subcore_barrier()

        for m in range(LEAD):
            pltpu.async_copy(u_sh.at[sidx.at[m]], rows.at[m], gsem[m])

        def step(jj, carry):
            for k in range(NSLOT):
                j = jj * NSLOT + k
                k6 = (k + LEAD) % NSLOT
                pltpu.make_async_copy(u_sh.at[sidx.at[j]], rows.at[k],
                                      gsem[k]).wait()
                pltpu.async_copy(rows.at[k], acc_sh.at[didx.at[j]], rsem[k],
                                 add=True)

                @pl.when(j >= LEAD)
                def _():
                    pltpu.make_async_copy(rows.at[k6], acc_sh.at[didx.at[0]],
                                          rsem[k6]).wait()

                @pl.when(j + LEAD < NBT)
                def _():
                    pltpu.async_copy(u_sh.at[sidx.at[j + LEAD]], rows.at[k6],
                                     gsem[k6])
                if with_cnt:
                    mine = lax.select(cid == 0, j < CNT_SPLIT, j >= CNT_SPLIT)

                    @pl.when(mine)
                    def _():
                        pltpu.sync_copy(ones_v, cnt_sh.at[didx.at[j]],
                                        add=True)
            return carry

        lax.fori_loop(0, NBT // NSLOT, step, 0)
        for i in range(LEAD):
            k = (NBT - LEAD + i) % NSLOT
            pltpu.make_async_copy(rows.at[k], acc_sh.at[didx.at[0]],
                                  rsem[k]).wait()

        @pl.when(has_tail)
        def _():
            pltpu.sync_copy(u_sh.at[sidx.at[NBT]], rows.at[0])
            pltpu.sync_copy(rows.at[0], acc_sh.at[didx.at[NBT]], add=True)
            if with_cnt:
                @pl.when(cid == 1)
                def _():
                    pltpu.sync_copy(ones_v, cnt_sh.at[didx.at[NBT]],
                                    add=True)

        plsc.subcore_barrier()

        rs = pl.ds(sid * RZ, RZ)

        @pl.when(cid == 0)
        def _():
            pltpu.sync_copy(acc_sh.at[rs], agg_lo_hbm.at[rs])
            if with_cnt:
                pltpu.sync_copy(cnt_sh.at[rs], cnt0_hbm.at[rs])

        @pl.when(cid == 1)
        def _():
            pltpu.sync_copy(acc_sh.at[rs], agg_hi_hbm.at[rs])
            if with_cnt:
                pltpu.sync_copy(cnt_sh.at[rs], cnt1_hbm.at[rs])

    kern = pl.kernel(body, out_type=out_type, mesh=_mesh(), scratch_types=scratch,
                     compiler_params=pltpu.CompilerParams(
                         use_tc_tiling_on_sc=False, needs_layout_passes=False))
    if with_cnt:
        return kern(u_lo, u_hi, src_b, dst_b, z32, z8, ones8)
    return kern(u_lo, u_hi, src_b, dst_b, z32)


def _sc_score(pq, srcl, dstl):
    def body(pq_hbm, srcl_hbm, dstl_hbm, out_hbm,
             pq_v, si_v, di_v, out_v):
        cid = lax.axis_index("c")
        sid = lax.axis_index("s")
        wid = cid * NS + sid
        base = jnp.minimum(wid * LT, LBL - LT)
        pltpu.sync_copy(pq_hbm, pq_v)
        pltpu.sync_copy(srcl_hbm.at[pl.ds(base, LT)], si_v)
        pltpu.sync_copy(dstl_hbm.at[pl.ds(base, LT)], di_v)
        col0 = jnp.zeros((16,), jnp.int32)
        col1 = jnp.ones((16,), jnp.int32)

        def step(t, carry):
            sl = pl.ds(t * 16, 16)
            pv = plsc.load_gather(pq_v, [si_v[sl], col0])
            qv = plsc.load_gather(pq_v, [di_v[sl], col1])
            out_v[sl] = pv + qv
            return carry

        lax.fori_loop(0, LT // 16, step, 0)
        pltpu.sync_copy(out_v, out_hbm.at[pl.ds(base, LT)])

    kern = pl.kernel(
        body,
        out_type=jax.ShapeDtypeStruct((LBL,), jnp.float32),
        mesh=_mesh(),
        scratch_types=[pltpu.VMEM((N, 2), jnp.float32),
                       pltpu.VMEM((LT,), jnp.int32),
                       pltpu.VMEM((LT,), jnp.int32),
                       pltpu.VMEM((LT,), jnp.float32)],
        compiler_params=pltpu.CompilerParams(use_tc_tiling_on_sc=False,
                                             needs_layout_passes=False))
    return kern(pq, srcl, dstl)


_BR = 1000


def _dense1(x, W_l, W_r, b):
    def body(x_ref, wl_ref, wr_ref, b_ref, u_lo_ref, u_hi_ref, r_ref):
        xb = x_ref[...]
        u = jnp.dot(xb, wl_ref[...], preferred_element_type=jnp.float32)
        u_lo_ref[...] = u[:, :32]
        u_hi_ref[...] = u[:, 32:]
        r_ref[...] = (jnp.dot(xb, wr_ref[...], preferred_element_type=jnp.float32)
                      + b_ref[...])

    return pl.pallas_call(
        body,
        grid=(N // _BR,),
        in_specs=[pl.BlockSpec((_BR, 128), lambda i: (i, 0)),
                  pl.BlockSpec((128, 64), lambda i: (0, 0)),
                  pl.BlockSpec((128, 64), lambda i: (0, 0)),
                  pl.BlockSpec((1, 64), lambda i: (0, 0))],
        out_specs=[pl.BlockSpec((_BR, 32), lambda i: (i, 0)),
                   pl.BlockSpec((_BR, 32), lambda i: (i, 0)),
                   pl.BlockSpec((_BR, 64), lambda i: (i, 0))],
        out_shape=[jax.ShapeDtypeStruct((N, 32), jnp.float32),
                   jax.ShapeDtypeStruct((N, 32), jnp.float32),
                   jax.ShapeDtypeStruct((N, 64), jnp.float32)],
    )(x, W_l, W_r, b)


def _dense2(a0, a1, c0, c1, r1, W_l, W_r, b):
    def body(a0_ref, a1_ref, c0_ref, c1_ref, r1_ref, wl_ref, wr_ref, b_ref,
             u_lo_ref, u_hi_ref, r_ref, ci_ref):
        cnt = c0_ref[...][:, 0:1] + c1_ref[...][:, 0:1]
        ci = 1.0 / jnp.maximum(cnt, 1.0)
        mean = jnp.concatenate([a0_ref[...], a1_ref[...]], axis=1) * ci
        h = jnp.maximum(mean + r1_ref[...], 0.0)
        u = jnp.dot(h, wl_ref[...], preferred_element_type=jnp.float32)
        u_lo_ref[...] = u[:, :32]
        u_hi_ref[...] = u[:, 32:]
        r_ref[...] = (jnp.dot(h, wr_ref[...], preferred_element_type=jnp.float32)
                      + b_ref[...])
        ci_ref[...] = ci

    return pl.pallas_call(
        body,
        grid=(N // _BR,),
        in_specs=[pl.BlockSpec((_BR, 32), lambda i: (i, 0)),
                  pl.BlockSpec((_BR, 32), lambda i: (i, 0)),
                  pl.BlockSpec((_BR, CW), lambda i: (i, 0)),
                  pl.BlockSpec((_BR, CW), lambda i: (i, 0)),
                  pl.BlockSpec((_BR, 64), lambda i: (i, 0)),
                  pl.BlockSpec((64, 64), lambda i: (0, 0)),
                  pl.BlockSpec((64, 64), lambda i: (0, 0)),
                  pl.BlockSpec((1, 64), lambda i: (0, 0))],
        out_specs=[pl.BlockSpec((_BR, 32), lambda i: (i, 0)),
                   pl.BlockSpec((_BR, 32), lambda i: (i, 0)),
                   pl.BlockSpec((_BR, 64), lambda i: (i, 0)),
                   pl.BlockSpec((_BR, 1), lambda i: (i, 0))],
        out_shape=[jax.ShapeDtypeStruct((N, 32), jnp.float32),
                   jax.ShapeDtypeStruct((N, 32), jnp.float32),
                   jax.ShapeDtypeStruct((N, 64), jnp.float32),
                   jax.ShapeDtypeStruct((N, 1), jnp.float32)],
    )(a0, a1, c0, c1, r1, W_l, W_r, b)


def _dense3(a0, a1, ci, r2, Wsr, bsr):
    def body(a0_ref, a1_ref, ci_ref, r2_ref, ws_ref, bs_ref, pq_ref):
        mean = jnp.concatenate([a0_ref[...], a1_ref[...]], axis=1) * ci_ref[...]
        h = jnp.maximum(mean + r2_ref[...], 0.0)
        pq_ref[...] = (jnp.dot(h, ws_ref[...], preferred_element_type=jnp.float32)
                       + bs_ref[...])

    return pl.pallas_call(
        body,
        grid=(N // _BR,),
        in_specs=[pl.BlockSpec((_BR, 32), lambda i: (i, 0)),
                  pl.BlockSpec((_BR, 32), lambda i: (i, 0)),
                  pl.BlockSpec((_BR, 1), lambda i: (i, 0)),
                  pl.BlockSpec((_BR, 64), lambda i: (i, 0)),
                  pl.BlockSpec((64, 2), lambda i: (0, 0)),
                  pl.BlockSpec((1, 2), lambda i: (0, 0))],
        out_specs=pl.BlockSpec((_BR, 2), lambda i: (i, 0)),
        out_shape=jax.ShapeDtypeStruct((N, 2), jnp.float32),
    )(a0, a1, ci, r2, Wsr, bsr)


def kernel(x, edge_index, edge_label_index, W_l1, W_r1, b1, W_l2, W_r2, b2,
           Ws, bs):
    src_b = edge_index[0].astype(jnp.int32).reshape(NB, EB)
    dst_b = edge_index[1].astype(jnp.int32).reshape(NB, EB)
    srcl = edge_label_index[0].astype(jnp.int32)
    dstl = edge_label_index[1].astype(jnp.int32)

    z32 = jnp.zeros((NPAD, 32), jnp.float32)
    z8 = jnp.zeros((NPAD, CW), jnp.float32)
    ones8 = jnp.ones((EB, CW), jnp.float32)

    u1lo, u1hi, r1 = _dense1(x, W_l1, W_r1, b1.reshape(1, 64))
    a0, a1, c0, c1 = _sc_agg(u1lo, u1hi, src_b, dst_b, z32, z8, ones8, True)
    u2lo, u2hi, r2, ci = _dense2(a0, a1, c0, c1, r1, W_l2, W_r2,
                                 b2.reshape(1, 64))
    b0, b1_ = _sc_agg(u2lo, u2hi, src_b, dst_b, z32, None, None, False)
    Wsr = jnp.concatenate([Ws[:64], Ws[64:]], axis=1)
    bsr = jnp.stack([bs[0], jnp.zeros((), jnp.float32)]).reshape(1, 2)
    pq = _dense3(b0, b1_, ci, r2, Wsr, bsr)
    return _sc_score(pq, srcl, dstl)


# speedup vs baseline: 15.9183x; 1.0387x over previous
import functools

import jax
import jax.numpy as jnp
from jax import lax
from jax.experimental import pallas as pl
from jax.experimental.pallas import tpu as pltpu
from jax.experimental.pallas import tpu_sc as plsc

N = 10000
E = 320000
LBL = 100000
NC, NS = 2, 16
NW = NC * NS
EB = 128
J = 80
E_PAD = NW * J * EB
J2 = 160
LT = 3136
L_PAD = NW * LT
NPAD = 10112
RZ = NPAD // NS
CW = 8
NBUF = 4

@functools.cache
def _mesh():
    return plsc.VectorSubcoreMesh(core_axis_name="c", subcore_axis_name="s",
                                  num_cores=NC, num_subcores=NS)


NB = 2500
NBT = 156
CNT_SPLIT = NBT // 2
NSLOT = 6
LEAD = 3


def _sc_agg(u_lo, u_hi, eidx, z32, z8, ones8, with_cnt):
    out_type = [jax.ShapeDtypeStruct((NPAD, 32), jnp.float32),
                jax.ShapeDtypeStruct((NPAD, 32), jnp.float32)]
    scratch = [pltpu.VMEM((NBT + 1, EB), jnp.int32),
               pltpu.VMEM((NBT + 1, EB), jnp.int32),
               pltpu.VMEM((NSLOT, EB, 32), jnp.float32),
               pltpu.VMEM_SHARED((NPAD, 32), jnp.float32),
               pltpu.VMEM_SHARED((N, 32), jnp.float32),
               *[pltpu.SemaphoreType.DMA] * (2 * NSLOT)]
    if with_cnt:
        out_type += [jax.ShapeDtypeStruct((NPAD, CW), jnp.float32),
                     jax.ShapeDtypeStruct((NPAD, CW), jnp.float32)]
        scratch += [pltpu.VMEM((EB, CW), jnp.float32),
                    pltpu.VMEM_SHARED((NPAD, CW), jnp.float32)]

    def body(*refs):
        if with_cnt:
            (ulo_hbm, uhi_hbm, eidx_hbm, z32_hbm, z8_hbm, ones_hbm,
             agg_lo_hbm, agg_hi_hbm, cnt0_hbm, cnt1_hbm,
             sidx, didx, rows, acc_sh, u_sh, *sems) = refs[:15 + 2 * NSLOT]
            ones_v, cnt_sh = refs[15 + 2 * NSLOT:]
        else:
            (ulo_hbm, uhi_hbm, eidx_hbm, z32_hbm,
             agg_lo_hbm, agg_hi_hbm,
             sidx, didx, rows, acc_sh, u_sh, *sems) = refs
        gsem = list(sems[:NSLOT])
        rsem = list(sems[NSLOT:2 * NSLOT])
        cid = lax.axis_index("c")
        sid = lax.axis_index("s")
        has_tail = sid < NB - NS * NBT

        pltpu.sync_copy(z32_hbm.at[pl.ds(sid * RZ, RZ)],
                        acc_sh.at[pl.ds(sid * RZ, RZ)])
        us = pl.ds(sid * (N // NS), N // NS)

        @pl.when(cid == 0)
        def _():
            pltpu.sync_copy(ulo_hbm.at[us], u_sh.at[us])

        @pl.when(cid == 1)
        def _():
            pltpu.sync_copy(uhi_hbm.at[us], u_sh.at[us])

        bs_ = pl.ds(sid * NBT, NBT)
        pltpu.sync_copy(eidx_hbm.at[0, bs_], sidx.at[pl.ds(0, NBT)])
        pltpu.sync_copy(eidx_hbm.at[1, bs_], didx.at[pl.ds(0, NBT)])

        @pl.when(has_tail)
        def _():
            ts_ = pl.ds(NS * NBT + sid, 1)
            pltpu.sync_copy(eidx_hbm.at[0, ts_], sidx.at[pl.ds(NBT, 1)])
            pltpu.sync_copy(eidx_hbm.at[1, ts_], didx.at[pl.ds(NBT, 1)])

        if with_cnt:
            pltpu.sync_copy(z8_hbm.at[pl.ds(sid * RZ, RZ)],
                            cnt_sh.at[pl.ds(sid * RZ, RZ)])
            pltpu.sync_copy(ones_hbm, ones_v)
        plsc.subcore_barrier()

        for m in range(LEAD):
            pltpu.async_copy(u_sh.at[sidx.at[m]], rows.at[m], gsem[m])

        def step(jj, carry):
            for k in range(NSLOT):
                j = jj * NSLOT + k
                k6 = (k + LEAD) % NSLOT
                pltpu.make_async_copy(u_sh.at[sidx.at[j]], rows.at[k],
                                      gsem[k]).wait()
                pltpu.async_copy(rows.at[k], acc_sh.at[didx.at[j]], rsem[k],
                                 add=True)

                @pl.when(j >= LEAD)
                def _():
                    pltpu.make_async_copy(rows.at[k6], acc_sh.at[didx.at[0]],
                                          rsem[k6]).wait()

                @pl.when(j + LEAD < NBT)
                def _():
                    pltpu.async_copy(u_sh.at[sidx.at[j + LEAD]], rows.at[k6],
                                     gsem[k6])
                if with_cnt:
                    mine = lax.select(cid == 0, j < CNT_SPLIT, j >= CNT_SPLIT)

                    @pl.when(mine)
                    def _():
                        pltpu.sync_copy(ones_v, cnt_sh.at[didx.at[j]],
                                        add=True)
            return carry

        lax.fori_loop(0, NBT // NSLOT, step, 0)
        for i in range(LEAD):
            k = (NBT - LEAD + i) % NSLOT
            pltpu.make_async_copy(rows.at[k], acc_sh.at[didx.at[0]],
                                  rsem[k]).wait()

        @pl.when(has_tail)
        def _():
            pltpu.sync_copy(u_sh.at[sidx.at[NBT]], rows.at[0])
            pltpu.sync_copy(rows.at[0], acc_sh.at[didx.at[NBT]], add=True)
            if with_cnt:
                @pl.when(cid == 1)
                def _():
                    pltpu.sync_copy(ones_v, cnt_sh.at[didx.at[NBT]],
                                    add=True)

        plsc.subcore_barrier()

        rs = pl.ds(sid * RZ, RZ)

        @pl.when(cid == 0)
        def _():
            pltpu.sync_copy(acc_sh.at[rs], agg_lo_hbm.at[rs])
            if with_cnt:
                pltpu.sync_copy(cnt_sh.at[rs], cnt0_hbm.at[rs])

        @pl.when(cid == 1)
        def _():
            pltpu.sync_copy(acc_sh.at[rs], agg_hi_hbm.at[rs])
            if with_cnt:
                pltpu.sync_copy(cnt_sh.at[rs], cnt1_hbm.at[rs])

    kern = pl.kernel(body, out_type=out_type, mesh=_mesh(), scratch_types=scratch,
                     compiler_params=pltpu.CompilerParams(
                         use_tc_tiling_on_sc=False, needs_layout_passes=False))
    if with_cnt:
        return kern(u_lo, u_hi, eidx, z32, z8, ones8)
    return kern(u_lo, u_hi, eidx, z32)


def _sc_score(pq, elbl):
    def body(pq_hbm, elbl_hbm, out_hbm,
             pq_v, si_v, di_v, out_v):
        cid = lax.axis_index("c")
        sid = lax.axis_index("s")
        wid = cid * NS + sid
        base = jnp.minimum(wid * LT, LBL - LT)
        pltpu.sync_copy(pq_hbm, pq_v)
        pltpu.sync_copy(elbl_hbm.at[0, pl.ds(base, LT)], si_v)
        pltpu.sync_copy(elbl_hbm.at[1, pl.ds(base, LT)], di_v)
        col0 = jnp.zeros((16,), jnp.int32)
        col1 = jnp.ones((16,), jnp.int32)

        def step(t, carry):
            sl = pl.ds(t * 16, 16)
            pv = plsc.load_gather(pq_v, [si_v[sl], col0])
            qv = plsc.load_gather(pq_v, [di_v[sl], col1])
            out_v[sl] = pv + qv
            return carry

        lax.fori_loop(0, LT // 16, step, 0)
        pltpu.sync_copy(out_v, out_hbm.at[pl.ds(base, LT)])

    kern = pl.kernel(
        body,
        out_type=jax.ShapeDtypeStruct((LBL,), jnp.float32),
        mesh=_mesh(),
        scratch_types=[pltpu.VMEM((N, 2), jnp.float32),
                       pltpu.VMEM((LT,), jnp.int32),
                       pltpu.VMEM((LT,), jnp.int32),
                       pltpu.VMEM((LT,), jnp.float32)],
        compiler_params=pltpu.CompilerParams(use_tc_tiling_on_sc=False,
                                             needs_layout_passes=False))
    return kern(pq, elbl)


_BR = 1000


def _dense1(x, W_l, W_r, b):
    def body(x_ref, wl_ref, wr_ref, b_ref, u_lo_ref, u_hi_ref, r_ref):
        xb = x_ref[...]
        u = jnp.dot(xb, wl_ref[...], preferred_element_type=jnp.float32)
        u_lo_ref[...] = u[:, :32]
        u_hi_ref[...] = u[:, 32:]
        r_ref[...] = (jnp.dot(xb, wr_ref[...], preferred_element_type=jnp.float32)
                      + b_ref[...])

    return pl.pallas_call(
        body,
        grid=(N // _BR,),
        in_specs=[pl.BlockSpec((_BR, 128), lambda i: (i, 0)),
                  pl.BlockSpec((128, 64), lambda i: (0, 0)),
                  pl.BlockSpec((128, 64), lambda i: (0, 0)),
                  pl.BlockSpec((1, 64), lambda i: (0, 0))],
        out_specs=[pl.BlockSpec((_BR, 32), lambda i: (i, 0)),
                   pl.BlockSpec((_BR, 32), lambda i: (i, 0)),
                   pl.BlockSpec((_BR, 64), lambda i: (i, 0))],
        out_shape=[jax.ShapeDtypeStruct((N, 32), jnp.float32),
                   jax.ShapeDtypeStruct((N, 32), jnp.float32),
                   jax.ShapeDtypeStruct((N, 64), jnp.float32)],
    )(x, W_l, W_r, b)


def _dense2(a0, a1, c0, c1, r1, W_l, W_r, b):
    def body(a0_ref, a1_ref, c0_ref, c1_ref, r1_ref, wl_ref, wr_ref, b_ref,
             u_lo_ref, u_hi_ref, r_ref, ci_ref):
        cnt = c0_ref[...][:, 0:1] + c1_ref[...][:, 0:1]
        ci = 1.0 / jnp.maximum(cnt, 1.0)
        mean = jnp.concatenate([a0_ref[...], a1_ref[...]], axis=1) * ci
        h = jnp.maximum(mean + r1_ref[...], 0.0)
        u = jnp.dot(h, wl_ref[...], preferred_element_type=jnp.float32)
        u_lo_ref[...] = u[:, :32]
        u_hi_ref[...] = u[:, 32:]
        r_ref[...] = (jnp.dot(h, wr_ref[...], preferred_element_type=jnp.float32)
                      + b_ref[...])
        ci_ref[...] = ci

    return pl.pallas_call(
        body,
        grid=(N // _BR,),
        in_specs=[pl.BlockSpec((_BR, 32), lambda i: (i, 0)),
                  pl.BlockSpec((_BR, 32), lambda i: (i, 0)),
                  pl.BlockSpec((_BR, CW), lambda i: (i, 0)),
                  pl.BlockSpec((_BR, CW), lambda i: (i, 0)),
                  pl.BlockSpec((_BR, 64), lambda i: (i, 0)),
                  pl.BlockSpec((64, 64), lambda i: (0, 0)),
                  pl.BlockSpec((64, 64), lambda i: (0, 0)),
                  pl.BlockSpec((1, 64), lambda i: (0, 0))],
        out_specs=[pl.BlockSpec((_BR, 32), lambda i: (i, 0)),
                   pl.BlockSpec((_BR, 32), lambda i: (i, 0)),
                   pl.BlockSpec((_BR, 64), lambda i: (i, 0)),
                   pl.BlockSpec((_BR, 1), lambda i: (i, 0))],
        out_shape=[jax.ShapeDtypeStruct((N, 32), jnp.float32),
                   jax.ShapeDtypeStruct((N, 32), jnp.float32),
                   jax.ShapeDtypeStruct((N, 64), jnp.float32),
                   jax.ShapeDtypeStruct((N, 1), jnp.float32)],
    )(a0, a1, c0, c1, r1, W_l, W_r, b)


def _dense3(a0, a1, ci, r2, Wsr, bsr):
    def body(a0_ref, a1_ref, ci_ref, r2_ref, ws_ref, bs_ref, pq_ref):
        mean = jnp.concatenate([a0_ref[...], a1_ref[...]], axis=1) * ci_ref[...]
        h = jnp.maximum(mean + r2_ref[...], 0.0)
        pq_ref[...] = (jnp.dot(h, ws_ref[...], preferred_element_type=jnp.float32)
                       + bs_ref[...])

    return pl.pallas_call(
        body,
        grid=(N // _BR,),
        in_specs=[pl.BlockSpec((_BR, 32), lambda i: (i, 0)),
                  pl.BlockSpec((_BR, 32), lambda i: (i, 0)),
                  pl.BlockSpec((_BR, 1), lambda i: (i, 0)),
                  pl.BlockSpec((_BR, 64), lambda i: (i, 0)),
                  pl.BlockSpec((64, 2), lambda i: (0, 0)),
                  pl.BlockSpec((1, 2), lambda i: (0, 0))],
        out_specs=pl.BlockSpec((_BR, 2), lambda i: (i, 0)),
        out_shape=jax.ShapeDtypeStruct((N, 2), jnp.float32),
    )(a0, a1, ci, r2, Wsr, bsr)


def kernel(x, edge_index, edge_label_index, W_l1, W_r1, b1, W_l2, W_r2, b2,
           Ws, bs):
    eidx = edge_index.astype(jnp.int32).reshape(2, NB, EB)
    elbl = edge_label_index.astype(jnp.int32)

    z32 = jnp.zeros((NPAD, 32), jnp.float32)
    z8 = jnp.zeros((NPAD, CW), jnp.float32)
    ones8 = jnp.ones((EB, CW), jnp.float32)

    u1lo, u1hi, r1 = _dense1(x, W_l1, W_r1, b1.reshape(1, 64))
    a0, a1, c0, c1 = _sc_agg(u1lo, u1hi, eidx, z32, z8, ones8, True)
    u2lo, u2hi, r2, ci = _dense2(a0, a1, c0, c1, r1, W_l2, W_r2,
                                 b2.reshape(1, 64))
    b0, b1_ = _sc_agg(u2lo, u2hi, eidx, z32, None, None, False)
    Wsr = jnp.concatenate([Ws[:64], Ws[64:]], axis=1)
    bsr = jnp.stack([bs[0], jnp.zeros((), jnp.float32)]).reshape(1, 2)
    pq = _dense3(b0, b1_, ci, r2, Wsr, bsr)
    return _sc_score(pq, elbl)


# speedup vs baseline: 16.0976x; 1.0113x over previous
import functools

import jax
import jax.numpy as jnp
from jax import lax
from jax.experimental import pallas as pl
from jax.experimental.pallas import tpu as pltpu
from jax.experimental.pallas import tpu_sc as plsc

N = 10000
E = 320000
LBL = 100000
NC, NS = 2, 16
NW = NC * NS
EB = 128
J = 80
E_PAD = NW * J * EB
J2 = 160
LT = 3136
L_PAD = NW * LT
NPAD = 10112
RZ = NPAD // NS
CW = 8
NBUF = 4

@functools.cache
def _mesh():
    return plsc.VectorSubcoreMesh(core_axis_name="c", subcore_axis_name="s",
                                  num_cores=NC, num_subcores=NS)


NB = 2500
NBT = 156
CNT_SPLIT = NBT // 2
NSLOT = 6
LEAD = 3


def _sc_agg(u_lo, u_hi, eidx, z32, z8, ones8, with_cnt):
    out_type = [jax.ShapeDtypeStruct((NPAD, 32), jnp.float32),
                jax.ShapeDtypeStruct((NPAD, 32), jnp.float32)]
    scratch = [pltpu.VMEM((NBT + 1, EB), jnp.int32),
               pltpu.VMEM((NBT + 1, EB), jnp.int32),
               pltpu.VMEM((NSLOT, EB, 32), jnp.float32),
               pltpu.VMEM_SHARED((NPAD, 32), jnp.float32),
               pltpu.VMEM_SHARED((N, 32), jnp.float32),
               *[pltpu.SemaphoreType.DMA] * (2 * NSLOT)]
    if with_cnt:
        out_type += [jax.ShapeDtypeStruct((NPAD, CW), jnp.float32),
                     jax.ShapeDtypeStruct((NPAD, CW), jnp.float32)]
        scratch += [pltpu.VMEM((EB, CW), jnp.float32),
                    pltpu.VMEM_SHARED((NPAD, CW), jnp.float32),
                    *[pltpu.SemaphoreType.DMA] * NSLOT]

    def body(*refs):
        if with_cnt:
            (ulo_hbm, uhi_hbm, eidx_hbm, z32_hbm, z8_hbm, ones_hbm,
             agg_lo_hbm, agg_hi_hbm, cnt0_hbm, cnt1_hbm,
             sidx, didx, rows, acc_sh, u_sh, *sems) = refs[:15 + 2 * NSLOT]
            ones_v, cnt_sh, *csem = refs[15 + 2 * NSLOT:]
        else:
            (ulo_hbm, uhi_hbm, eidx_hbm, z32_hbm,
             agg_lo_hbm, agg_hi_hbm,
             sidx, didx, rows, acc_sh, u_sh, *sems) = refs
        gsem = list(sems[:NSLOT])
        rsem = list(sems[NSLOT:2 * NSLOT])
        cid = lax.axis_index("c")
        sid = lax.axis_index("s")
        has_tail = sid < NB - NS * NBT

        pltpu.sync_copy(z32_hbm.at[pl.ds(sid * RZ, RZ)],
                        acc_sh.at[pl.ds(sid * RZ, RZ)])
        us = pl.ds(sid * (N // NS), N // NS)

        @pl.when(cid == 0)
        def _():
            pltpu.sync_copy(ulo_hbm.at[us], u_sh.at[us])

        @pl.when(cid == 1)
        def _():
            pltpu.sync_copy(uhi_hbm.at[us], u_sh.at[us])

        bs_ = pl.ds(sid * NBT, NBT)
        pltpu.sync_copy(eidx_hbm.at[0, bs_], sidx.at[pl.ds(0, NBT)])
        pltpu.sync_copy(eidx_hbm.at[1, bs_], didx.at[pl.ds(0, NBT)])

        @pl.when(has_tail)
        def _():
            ts_ = pl.ds(NS * NBT + sid, 1)
            pltpu.sync_copy(eidx_hbm.at[0, ts_], sidx.at[pl.ds(NBT, 1)])
            pltpu.sync_copy(eidx_hbm.at[1, ts_], didx.at[pl.ds(NBT, 1)])

        if with_cnt:
            pltpu.sync_copy(z8_hbm.at[pl.ds(sid * RZ, RZ)],
                            cnt_sh.at[pl.ds(sid * RZ, RZ)])
            pltpu.sync_copy(ones_hbm, ones_v)
        plsc.subcore_barrier()

        for m in range(LEAD):
            pltpu.async_copy(u_sh.at[sidx.at[m]], rows.at[m], gsem[m])

        def step(jj, carry):
            for k in range(NSLOT):
                j = jj * NSLOT + k
                k6 = (k + LEAD) % NSLOT
                pltpu.make_async_copy(u_sh.at[sidx.at[j]], rows.at[k],
                                      gsem[k]).wait()
                pltpu.async_copy(rows.at[k], acc_sh.at[didx.at[j]], rsem[k],
                                 add=True)

                @pl.when(j >= LEAD)
                def _():
                    pltpu.make_async_copy(rows.at[k6], acc_sh.at[didx.at[0]],
                                          rsem[k6]).wait()

                @pl.when(j + LEAD < NBT)
                def _():
                    pltpu.async_copy(u_sh.at[sidx.at[j + LEAD]], rows.at[k6],
                                     gsem[k6])
                if with_cnt:
                    split0 = lax.select(cid == 0, 0, CNT_SPLIT)
                    mine = lax.select(cid == 0, j < CNT_SPLIT, j >= CNT_SPLIT)

                    @pl.when(jnp.logical_and(mine, j >= split0 + NSLOT))
                    def _():
                        pltpu.make_async_copy(ones_v, cnt_sh.at[didx.at[0]],
                                              csem[k]).wait()

                    @pl.when(mine)
                    def _():
                        pltpu.async_copy(ones_v, cnt_sh.at[didx.at[j]],
                                         csem[k], add=True)
            return carry

        lax.fori_loop(0, NBT // NSLOT, step, 0)
        if with_cnt:
            for k in range(NSLOT):
                pltpu.make_async_copy(ones_v, cnt_sh.at[didx.at[0]],
                                      csem[k]).wait()
        for i in range(LEAD):
            k = (NBT - LEAD + i) % NSLOT
            pltpu.make_async_copy(rows.at[k], acc_sh.at[didx.at[0]],
                                  rsem[k]).wait()

        @pl.when(has_tail)
        def _():
            pltpu.sync_copy(u_sh.at[sidx.at[NBT]], rows.at[0])
            pltpu.sync_copy(rows.at[0], acc_sh.at[didx.at[NBT]], add=True)
            if with_cnt:
                @pl.when(cid == 1)
                def _():
                    pltpu.sync_copy(ones_v, cnt_sh.at[didx.at[NBT]],
                                    add=True)

        plsc.subcore_barrier()

        rs = pl.ds(sid * RZ, RZ)

        @pl.when(cid == 0)
        def _():
            pltpu.sync_copy(acc_sh.at[rs], agg_lo_hbm.at[rs])
            if with_cnt:
                pltpu.sync_copy(cnt_sh.at[rs], cnt0_hbm.at[rs])

        @pl.when(cid == 1)
        def _():
            pltpu.sync_copy(acc_sh.at[rs], agg_hi_hbm.at[rs])
            if with_cnt:
                pltpu.sync_copy(cnt_sh.at[rs], cnt1_hbm.at[rs])

    kern = pl.kernel(body, out_type=out_type, mesh=_mesh(), scratch_types=scratch,
                     compiler_params=pltpu.CompilerParams(
                         use_tc_tiling_on_sc=False, needs_layout_passes=False))
    if with_cnt:
        return kern(u_lo, u_hi, eidx, z32, z8, ones8)
    return kern(u_lo, u_hi, eidx, z32)


def _sc_score(pq, elbl):
    def body(pq_hbm, elbl_hbm, out_hbm,
             pq_v, si_v, di_v, out_v):
        cid = lax.axis_index("c")
        sid = lax.axis_index("s")
        wid = cid * NS + sid
        base = jnp.minimum(wid * LT, LBL - LT)
        pltpu.sync_copy(pq_hbm, pq_v)
        pltpu.sync_copy(elbl_hbm.at[0, pl.ds(base, LT)], si_v)
        pltpu.sync_copy(elbl_hbm.at[1, pl.ds(base, LT)], di_v)
        col0 = jnp.zeros((16,), jnp.int32)
        col1 = jnp.ones((16,), jnp.int32)

        def step(t, carry):
            sl = pl.ds(t * 16, 16)
            pv = plsc.load_gather(pq_v, [si_v[sl], col0])
            qv = plsc.load_gather(pq_v, [di_v[sl], col1])
            out_v[sl] = pv + qv
            return carry

        lax.fori_loop(0, LT // 16, step, 0)
        pltpu.sync_copy(out_v, out_hbm.at[pl.ds(base, LT)])

    kern = pl.kernel(
        body,
        out_type=jax.ShapeDtypeStruct((LBL,), jnp.float32),
        mesh=_mesh(),
        scratch_types=[pltpu.VMEM((N, 2), jnp.float32),
                       pltpu.VMEM((LT,), jnp.int32),
                       pltpu.VMEM((LT,), jnp.int32),
                       pltpu.VMEM((LT,), jnp.float32)],
        compiler_params=pltpu.CompilerParams(use_tc_tiling_on_sc=False,
                                             needs_layout_passes=False))
    return kern(pq, elbl)


_BR = 1000


def _dense1(x, W_l, W_r, b):
    def body(x_ref, wl_ref, wr_ref, b_ref, u_lo_ref, u_hi_ref, r_ref):
        xb = x_ref[...]
        u = jnp.dot(xb, wl_ref[...], preferred_element_type=jnp.float32)
        u_lo_ref[...] = u[:, :32]
        u_hi_ref[...] = u[:, 32:]
        r_ref[...] = (jnp.dot(xb, wr_ref[...], preferred_element_type=jnp.float32)
                      + b_ref[...])

    return pl.pallas_call(
        body,
        grid=(N // _BR,),
        in_specs=[pl.BlockSpec((_BR, 128), lambda i: (i, 0)),
                  pl.BlockSpec((128, 64), lambda i: (0, 0)),
                  pl.BlockSpec((128, 64), lambda i: (0, 0)),
                  pl.BlockSpec((1, 64), lambda i: (0, 0))],
        out_specs=[pl.BlockSpec((_BR, 32), lambda i: (i, 0)),
                   pl.BlockSpec((_BR, 32), lambda i: (i, 0)),
                   pl.BlockSpec((_BR, 64), lambda i: (i, 0))],
        out_shape=[jax.ShapeDtypeStruct((N, 32), jnp.float32),
                   jax.ShapeDtypeStruct((N, 32), jnp.float32),
                   jax.ShapeDtypeStruct((N, 64), jnp.float32)],
    )(x, W_l, W_r, b)


def _dense2(a0, a1, c0, c1, r1, W_l, W_r, b):
    def body(a0_ref, a1_ref, c0_ref, c1_ref, r1_ref, wl_ref, wr_ref, b_ref,
             u_lo_ref, u_hi_ref, r_ref, ci_ref):
        cnt = c0_ref[...][:, 0:1] + c1_ref[...][:, 0:1]
        ci = 1.0 / jnp.maximum(cnt, 1.0)
        mean = jnp.concatenate([a0_ref[...], a1_ref[...]], axis=1) * ci
        h = jnp.maximum(mean + r1_ref[...], 0.0)
        u = jnp.dot(h, wl_ref[...], preferred_element_type=jnp.float32)
        u_lo_ref[...] = u[:, :32]
        u_hi_ref[...] = u[:, 32:]
        r_ref[...] = (jnp.dot(h, wr_ref[...], preferred_element_type=jnp.float32)
                      + b_ref[...])
        ci_ref[...] = ci

    return pl.pallas_call(
        body,
        grid=(N // _BR,),
        in_specs=[pl.BlockSpec((_BR, 32), lambda i: (i, 0)),
                  pl.BlockSpec((_BR, 32), lambda i: (i, 0)),
                  pl.BlockSpec((_BR, CW), lambda i: (i, 0)),
                  pl.BlockSpec((_BR, CW), lambda i: (i, 0)),
                  pl.BlockSpec((_BR, 64), lambda i: (i, 0)),
                  pl.BlockSpec((64, 64), lambda i: (0, 0)),
                  pl.BlockSpec((64, 64), lambda i: (0, 0)),
                  pl.BlockSpec((1, 64), lambda i: (0, 0))],
        out_specs=[pl.BlockSpec((_BR, 32), lambda i: (i, 0)),
                   pl.BlockSpec((_BR, 32), lambda i: (i, 0)),
                   pl.BlockSpec((_BR, 64), lambda i: (i, 0)),
                   pl.BlockSpec((_BR, 1), lambda i: (i, 0))],
        out_shape=[jax.ShapeDtypeStruct((N, 32), jnp.float32),
                   jax.ShapeDtypeStruct((N, 32), jnp.float32),
                   jax.ShapeDtypeStruct((N, 64), jnp.float32),
                   jax.ShapeDtypeStruct((N, 1), jnp.float32)],
    )(a0, a1, c0, c1, r1, W_l, W_r, b)


def _dense3(a0, a1, ci, r2, Wsr, bsr):
    def body(a0_ref, a1_ref, ci_ref, r2_ref, ws_ref, bs_ref, pq_ref):
        mean = jnp.concatenate([a0_ref[...], a1_ref[...]], axis=1) * ci_ref[...]
        h = jnp.maximum(mean + r2_ref[...], 0.0)
        pq_ref[...] = (jnp.dot(h, ws_ref[...], preferred_element_type=jnp.float32)
                       + bs_ref[...])

    return pl.pallas_call(
        body,
        grid=(N // _BR,),
        in_specs=[pl.BlockSpec((_BR, 32), lambda i: (i, 0)),
                  pl.BlockSpec((_BR, 32), lambda i: (i, 0)),
                  pl.BlockSpec((_BR, 1), lambda i: (i, 0)),
                  pl.BlockSpec((_BR, 64), lambda i: (i, 0)),
                  pl.BlockSpec((64, 2), lambda i: (0, 0)),
                  pl.BlockSpec((1, 2), lambda i: (0, 0))],
        out_specs=pl.BlockSpec((_BR, 2), lambda i: (i, 0)),
        out_shape=jax.ShapeDtypeStruct((N, 2), jnp.float32),
    )(a0, a1, ci, r2, Wsr, bsr)


def kernel(x, edge_index, edge_label_index, W_l1, W_r1, b1, W_l2, W_r2, b2,
           Ws, bs):
    eidx = edge_index.astype(jnp.int32).reshape(2, NB, EB)
    elbl = edge_label_index.astype(jnp.int32)

    z32 = jnp.zeros((NPAD, 32), jnp.float32)
    z8 = jnp.zeros((NPAD, CW), jnp.float32)
    ones8 = jnp.ones((EB, CW), jnp.float32)

    u1lo, u1hi, r1 = _dense1(x, W_l1, W_r1, b1.reshape(1, 64))
    a0, a1, c0, c1 = _sc_agg(u1lo, u1hi, eidx, z32, z8, ones8, True)
    u2lo, u2hi, r2, ci = _dense2(a0, a1, c0, c1, r1, W_l2, W_r2,
                                 b2.reshape(1, 64))
    b0, b1_ = _sc_agg(u2lo, u2hi, eidx, z32, None, None, False)
    Wsr = jnp.concatenate([Ws[:64], Ws[64:]], axis=1)
    bsr = jnp.stack([bs[0], jnp.zeros((), jnp.float32)]).reshape(1, 2)
    pq = _dense3(b0, b1_, ci, r2, Wsr, bsr)
    return _sc_score(pq, elbl)
